# Initial kernel scaffold; baseline (speedup 1.0000x reference)
import jax
import jax.numpy as jnp
from jax.experimental import pallas as pl


def kernel(x_individual, x_family, params, edge_index_individual_child_of_family, edge_index_family_parent_of_individual, edge_index_individual_spouse_individual):
    raise NotImplementedError("write your pallas kernel here")


# speedup vs baseline: 7.6563x; 7.6563x over previous
import functools

import jax
import jax.numpy as jnp
from jax import lax
from jax.experimental import pallas as pl
from jax.experimental.pallas import tpu as pltpu
from jax.experimental.pallas import tpu_sc as plsc

F32 = jnp.float32
I32 = jnp.int32

D = 256
HA = 16
B_EDGE = 100000
NTILE = 32
TK = 3200
BP = NTILE * TK
NBLK = TK // 128
R_ROWS = 4096
KB2 = 64
BM = 512


def _mesh():
    return plsc.VectorSubcoreMesh(core_axis_name="c", subcore_axis_name="s")


def _iota16():
    return jax.lax.iota(I32, 16)


def _mm_stacked(x, w_stack, bias, relu):
    n = x.shape[0]
    s_chunks = w_stack.shape[0]
    mb = pl.cdiv(n, BM)

    def body(x_ref, w_ref, b_ref, o_ref):
        acc = jnp.dot(x_ref[...], w_ref[0], preferred_element_type=F32)
        acc = acc + b_ref[0]
        if relu:
            acc = jnp.maximum(acc, 0.0)
        o_ref[0] = acc

    return pl.pallas_call(
        body,
        grid=(mb, s_chunks),
        in_specs=[
            pl.BlockSpec((BM, D), lambda i, j: (i, 0)),
            pl.BlockSpec((1, D, D), lambda i, j: (j, 0, 0)),
            pl.BlockSpec((1, 1, D), lambda i, j: (j, 0, 0)),
        ],
        out_specs=pl.BlockSpec((1, BM, D), lambda i, j: (j, i, 0)),
        out_shape=jax.ShapeDtypeStruct((s_chunks, n, D), F32),
    )(x, w_stack, bias[:, None, :])


def _mm_thin(x, wa):
    n = x.shape[0]
    na = wa.shape[1]
    mb = pl.cdiv(n, BM)

    def body(x_ref, w_ref, o_ref):
        o_ref[...] = jnp.dot(x_ref[...], w_ref[...], preferred_element_type=F32)

    return pl.pallas_call(
        body,
        grid=(mb,),
        in_specs=[
            pl.BlockSpec((BM, D), lambda i: (i, 0)),
            pl.BlockSpec((D, na), lambda i: (0, 0)),
        ],
        out_specs=pl.BlockSpec((BM, na), lambda i: (i, 0)),
        out_shape=jax.ShapeDtypeStruct((n, na), F32),
    )(x, wa)


def _combine2(a, b, bias, n):
    mb = pl.cdiv(n, BM)

    def body(a_ref, b_ref, bias_ref, o_ref):
        o_ref[...] = jnp.maximum(a_ref[...] + b_ref[...] + bias_ref[...], 0.0)

    return pl.pallas_call(
        body,
        grid=(mb,),
        in_specs=[
            pl.BlockSpec((BM, D), lambda i: (i, 0)),
            pl.BlockSpec((BM, D), lambda i: (i, 0)),
            pl.BlockSpec((1, D), lambda i: (0, 0)),
        ],
        out_specs=pl.BlockSpec((BM, D), lambda i: (i, 0)),
        out_shape=jax.ShapeDtypeStruct((n, D), F32),
    )(a, b, bias)


def _combine1(a, bias, n):
    mb = pl.cdiv(n, BM)

    def body(a_ref, bias_ref, o_ref):
        o_ref[...] = jnp.maximum(a_ref[...] + bias_ref[...], 0.0)

    return pl.pallas_call(
        body,
        grid=(mb,),
        in_specs=[
            pl.BlockSpec((BM, D), lambda i: (i, 0)),
            pl.BlockSpec((1, D), lambda i: (0, 0)),
        ],
        out_specs=pl.BlockSpec((BM, D), lambda i: (i, 0)),
        out_shape=jax.ShapeDtypeStruct((n, D), F32),
    )(a, bias)


@functools.partial(jax.jit, static_argnums=(4,))
def _sc_pass1(a_src_tab, a_dst_tab, src, dst, nd_pad):
    zfull, ztail = divmod(nd_pad // 16, 128)

    def body(asrc_hbm, adst_hbm, src_hbm, dst_hbm, ex_hbm, s_hbm,
             srcbuf, dstbuf, arow, brow, exbuf, zbuf, sem, s_acc):
        cid = lax.axis_index("c")
        sid = lax.axis_index("s")
        iota = _iota16()
        zero16 = jnp.zeros((16,), F32)

        def zb(j, _):
            zbuf[j] = zero16
            return 0
        lax.fori_loop(0, 128, zb, 0)

        rpt = nd_pad // 16
        def zs(j, _):
            pltpu.sync_copy(zbuf, s_acc.at[pl.ds(sid * rpt + j * 128, 128)])
            return 0
        lax.fori_loop(0, zfull, zs, 0)
        if ztail:
            pltpu.sync_copy(zbuf.at[pl.ds(0, ztail)],
                            s_acc.at[pl.ds(sid * rpt + zfull * 128, ztail)])
        plsc.subcore_barrier()

        tile_base = (cid * 16 + sid) * TK

        def blk(bi, _):
            base = tile_base + bi * 128
            pltpu.sync_copy(src_hbm.at[pl.ds(base, 128)], srcbuf)
            pltpu.sync_copy(dst_hbm.at[pl.ds(base, 128)], dstbuf)
            pltpu.async_copy(asrc_hbm.at[srcbuf], arow, sem).wait()
            pltpu.async_copy(adst_hbm.at[dstbuf], brow, sem).wait()

            def ew(j, _):
                e = arow[j] + brow[j]
                e = jnp.where(e > 0, e, 0.2 * e)
                exbuf[j] = jnp.exp(e)
                return 0
            lax.fori_loop(0, 128, ew, 0)

            pltpu.sync_copy(exbuf, ex_hbm.at[pl.ds(base, 128)])
            pltpu.sync_copy(exbuf, s_acc.at[dstbuf], add=True)
            return 0
        lax.fori_loop(0, NBLK, blk, 0)
        plsc.subcore_barrier()

        for t in range(zfull):
            pltpu.sync_copy(s_acc.at[pl.ds(sid * rpt + t * 128, 128)],
                            s_hbm.at[cid].at[pl.ds(sid * rpt + t * 128, 128)])
        if ztail:
            pltpu.sync_copy(s_acc.at[pl.ds(sid * rpt + zfull * 128, ztail)],
                            s_hbm.at[cid].at[pl.ds(sid * rpt + zfull * 128, ztail)])

    kern = pl.kernel(
        body,
        out_type=(
            jax.ShapeDtypeStruct((BP, HA), F32),
            jax.ShapeDtypeStruct((2, nd_pad, HA), F32),
        ),
        mesh=_mesh(),
        scratch_types=[
            pltpu.VMEM((128,), I32),
            pltpu.VMEM((128,), I32),
            pltpu.VMEM((128, HA), F32),
            pltpu.VMEM((128, HA), F32),
            pltpu.VMEM((128, HA), F32),
            pltpu.VMEM((128, HA), F32),
            pltpu.SemaphoreType.DMA,
            pltpu.VMEM_SHARED((nd_pad, HA), F32),
        ],
        compiler_params=pltpu.CompilerParams(use_tc_tiling_on_sc=False),
    )
    return kern(a_src_tab, a_dst_tab, src, dst)


@functools.partial(jax.jit, static_argnums=())
def _sc_pass1b(ex, s_part, dst):
    def body(ex_hbm, s_hbm, dst_hbm, wpk_hbm,
             dstbuf, exrow, s0row, s1row, wbuf, sem):
        cid = lax.axis_index("c")
        sid = lax.axis_index("s")
        tile_base = (cid * 16 + sid) * TK

        def blk(bi, _):
            base = tile_base + bi * 128
            pltpu.sync_copy(dst_hbm.at[pl.ds(base, 128)], dstbuf)
            pltpu.sync_copy(ex_hbm.at[pl.ds(base, 128)], exrow)
            pltpu.async_copy(s_hbm.at[0].at[dstbuf], s0row, sem).wait()
            pltpu.async_copy(s_hbm.at[1].at[dstbuf], s1row, sem).wait()

            def ew(j, _):
                w = exrow[j] / (s0row[j] + s1row[j] + 1e-16)
                wbuf[j >> 3, pl.ds((j & 7) * 16, 16)] = w
                return 0
            lax.fori_loop(0, 128, ew, 0)

            pltpu.sync_copy(wbuf, wpk_hbm.at[pl.ds(base >> 3, 16)])
            return 0
        lax.fori_loop(0, NBLK, blk, 0)

    kern = pl.kernel(
        body,
        out_type=jax.ShapeDtypeStruct((BP // 8, 128), F32),
        mesh=_mesh(),
        scratch_types=[
            pltpu.VMEM((128,), I32),
            pltpu.VMEM((128, HA), F32),
            pltpu.VMEM((128, HA), F32),
            pltpu.VMEM((128, HA), F32),
            pltpu.VMEM((16, 128), F32),
            pltpu.SemaphoreType.DMA,
        ],
        compiler_params=pltpu.CompilerParams(use_tc_tiling_on_sc=False),
    )
    return kern(ex, s_part, dst)


@functools.partial(jax.jit, static_argnums=(4, 5, 6))
def _sc_pass2(h_stack, src, dst, wpk, slot, nd_pad, chead):
    nranges = nd_pad // R_ROWS
    rpt2 = R_ROWS // 16
    tk2 = TK * 2

    def body(h_hbm, src_hbm, dst_hbm, wpk_hbm, out_hbm,
             dstc, srcc, cb_src, cb_dst, cb_eid,
             gidx, widx, scidx, wtmp,
             rowbuf, wrow, zbuf, sem, acc):
        cid = lax.axis_index("c")
        sid = lax.axis_index("s")
        iota = _iota16()
        tile_base = sid * tk2
        zero16 = jnp.zeros((16,), F32)

        pltpu.sync_copy(dst_hbm.at[pl.ds(tile_base, tk2)], dstc)
        pltpu.sync_copy(src_hbm.at[pl.ds(tile_base, tk2)], srcc)

        def zb(k, _):
            for j in range(16):
                zbuf[k, pl.ds(j * 16, 16)] = zero16
            return 0
        lax.fori_loop(0, 16, zb, 0)

        def range_body(r, _):
            lo = r * R_ROWS

            @pl.when(lax.rem(r, 2) == cid)
            def _():
                for t in range(rpt2 // 16):
                    pltpu.sync_copy(zbuf, acc.at[pl.ds(sid * rpt2 + t * 16, 16)])
                plsc.subcore_barrier()

                def scan(j, cnt_v):
                    d = dstc[pl.ds(j * 16, 16)]
                    s = srcc[pl.ds(j * 16, 16)]
                    m = (d >= lo) & (d < lo + R_ROWS)
                    pos = jnp.where(m, cnt_v + plsc.cumsum(m.astype(I32)) - 1,
                                    tk2 + 8)
                    plsc.store_scatter(cb_src, [pos], s)
                    plsc.store_scatter(cb_dst, [pos], d)
                    eid = (tile_base + j * 16) + iota
                    plsc.store_scatter(cb_eid, [pos], eid)
                    return cnt_v + plsc.all_reduce_population_count(m)
                cnt_v = lax.fori_loop(0, tk2 // 16, scan,
                                      jnp.zeros((16,), I32))
                cnt = cnt_v[0]

                nb = (cnt + (KB2 - 1)) // KB2

                def proc(bb, _):
                    k0 = bb * KB2

                    def mkidx(v, _):
                        pos = k0 + v * 16
                        m = (pos + iota) < cnt
                        sv = cb_src[pl.ds(pos, 16)]
                        dv = cb_dst[pl.ds(pos, 16)]
                        ev = cb_eid[pl.ds(pos, 16)]
                        gidx[pl.ds(v * 16, 16)] = jnp.where(m, sv, 0)
                        widx[pl.ds(v * 16, 16)] = jnp.where(m, ev >> 3, 0)
                        scidx[pl.ds(v * 16, 16)] = jnp.where(m, dv - lo, R_ROWS)
                        return 0
                    lax.fori_loop(0, KB2 // 16, mkidx, 0)

                    pltpu.async_copy(h_hbm.at[slot].at[gidx], rowbuf, sem).wait()
                    pltpu.async_copy(wpk_hbm.at[widx], wrow, sem).wait()

                    def rowfn(e2, _):
                        sub = (cb_eid[pl.ds(k0 + e2, 16)][0] & 7) * 16
                        wtmp[...] = wrow[e2, pl.ds(sub, 16)]
                        for j in range(16):
                            hd = (16 * j) // chead
                            wsp = plsc.load_gather(wtmp,
                                                   [jnp.full((16,), hd, I32)])
                            rowbuf[e2, pl.ds(j * 16, 16)] = (
                                rowbuf[e2, pl.ds(j * 16, 16)] * wsp)
                        return 0
                    lax.fori_loop(0, KB2, rowfn, 0)

                    pltpu.sync_copy(rowbuf, acc.at[scidx], add=True)
                    return 0
                lax.fori_loop(0, nb, proc, 0)
                plsc.subcore_barrier()

                pltpu.sync_copy(acc.at[pl.ds(sid * rpt2, rpt2)],
                                out_hbm.at[pl.ds(lo + sid * rpt2, rpt2)])
            return 0
        lax.fori_loop(0, nranges, range_body, 0)

    kern = pl.kernel(
        body,
        out_type=jax.ShapeDtypeStruct((nd_pad, D), F32),
        mesh=_mesh(),
        scratch_types=[
            pltpu.VMEM((tk2,), I32),
            pltpu.VMEM((tk2,), I32),
            pltpu.VMEM((tk2 + 16,), I32),
            pltpu.VMEM((tk2 + 16,), I32),
            pltpu.VMEM((tk2 + 16,), I32),
            pltpu.VMEM((KB2,), I32),
            pltpu.VMEM((KB2,), I32),
            pltpu.VMEM((KB2,), I32),
            pltpu.VMEM((16,), F32),
            pltpu.VMEM((KB2, D), F32),
            pltpu.VMEM((KB2, 128), F32),
            pltpu.VMEM((16, D), F32),
            pltpu.SemaphoreType.DMA,
            pltpu.VMEM_SHARED((R_ROWS + 8, D), F32),
        ],
        compiler_params=pltpu.CompilerParams(use_tc_tiling_on_sc=False,
                                             needs_layout_passes=False),
    )
    return kern(h_stack, src, dst, wpk)


def _att_fold(p, heads, chead):
    wr = p["W"].reshape(D, heads, chead)
    a_s = jnp.einsum("khc,hc->kh", wr, p["att_src"],
                     precision=jax.lax.Precision.HIGHEST)
    a_d = jnp.einsum("khc,hc->kh", wr, p["att_dst"],
                     precision=jax.lax.Precision.HIGHEST)
    if heads < HA:
        a_s = jnp.pad(a_s, ((0, 0), (0, HA - heads)))
        a_d = jnp.pad(a_d, ((0, 0), (0, HA - heads)))
    return a_s, a_d


def _pad_rows(a, extra=8):
    return jnp.pad(a, ((0, extra), (0, 0)))


def _pad_edges(e, n_dst):
    src = e[0].astype(I32)
    dst = e[1].astype(I32)
    pad = BP - B_EDGE
    src = jnp.concatenate([src, jnp.zeros((pad,), I32)])
    dst = jnp.concatenate([dst, jnp.full((pad,), n_dst, I32)])
    return src, dst


def kernel(x_individual, x_family, params,
           edge_index_individual_child_of_family,
           edge_index_family_parent_of_individual,
           edge_index_individual_spouse_individual):
    n_ind = x_individual.shape[0]
    n_fam = x_family.shape[0]
    ndp_ind = ((n_ind + 8 + R_ROWS - 1) // R_ROWS) * R_ROWS
    ndp_fam = ((n_fam + 8 + R_ROWS - 1) // R_ROWS) * R_ROWS

    s1e, d1e = _pad_edges(edge_index_individual_child_of_family, n_fam)
    s2e, d2e = _pad_edges(edge_index_family_parent_of_individual, n_ind)
    s3e, d3e = _pad_edges(edge_index_individual_spouse_individual, n_ind)

    pe_i = params["emb"]["individual"]
    pe_f = params["emb"]["family"]
    x_i = _mm_stacked(x_individual, pe_i["W"][None], pe_i["b"][None], True)[0]
    x_f = _mm_stacked(x_family, pe_f["W"][None], pe_f["b"][None], True)[0]

    k1 = "individual__child_of__family"
    k2 = "family__parent_of__individual"
    k3 = "individual__spouse__individual"

    for l in range(4):
        concat = l < 3
        heads = 8 if concat else 1
        chead = D // heads
        lp = params["convs"][l]
        p1, p2, p3 = lp[k1], lp[k2], lp[k3]

        u_ind = _mm_stacked(x_i, jnp.stack([p1["W"], p3["W"]]),
                            jnp.zeros((2, D), F32), False)
        u_fam = _mm_stacked(x_f, p2["W"][None], jnp.zeros((1, D), F32), False)

        a1s, a1d = _att_fold(p1, heads, chead)
        a2s, a2d = _att_fold(p2, heads, chead)
        a3s, a3d = _att_fold(p3, heads, chead)
        wa_ind = jnp.concatenate([a1s, a2d, a3s, a3d], axis=1)
        wa_fam = jnp.concatenate([a1d, a2s], axis=1)
        ai = _mm_thin(x_i, wa_ind)
        af = _mm_thin(x_f, wa_fam)

        t1s = _pad_rows(ai[:, 0:16])
        t2d = _pad_rows(ai[:, 16:32])
        t3s = _pad_rows(ai[:, 32:48])
        t3d = _pad_rows(ai[:, 48:64])
        t1d = _pad_rows(af[:, 0:16])
        t2s = _pad_rows(af[:, 16:32])

        ex1, sp1 = _sc_pass1(t1s, t1d, s1e, d1e, ndp_fam)
        ex2, sp2 = _sc_pass1(t2s, t2d, s2e, d2e, ndp_ind)
        ex3, sp3 = _sc_pass1(t3s, t3d, s3e, d3e, ndp_ind)
        w1 = _sc_pass1b(ex1, sp1, d1e)
        w2 = _sc_pass1b(ex2, sp2, d2e)
        w3 = _sc_pass1b(ex3, sp3, d3e)

        o1 = _sc_pass2(u_ind, s1e, d1e, w1, 0, ndp_fam, chead)
        o2 = _sc_pass2(u_fam, s2e, d2e, w2, 0, ndp_ind, chead)
        o3 = _sc_pass2(u_ind, s3e, d3e, w3, 1, ndp_ind, chead)

        x_f = _combine1(o1, p1["bias"][None], n_fam)
        x_i = _combine2(o2, o3, (p2["bias"] + p3["bias"])[None], n_ind)

    pf = params["pred"]["father"]
    pm = params["pred"]["mother"]
    pred = _mm_stacked(x_i, jnp.stack([pf["W"], pm["W"]]),
                       jnp.stack([pf["b"], pm["b"]]), False)
    return (x_i, x_f, pred[0], pred[1])


# speedup vs baseline: 9.4455x; 1.2337x over previous
import functools

import jax
import jax.numpy as jnp
from jax import lax
from jax.experimental import pallas as pl
from jax.experimental.pallas import tpu as pltpu
from jax.experimental.pallas import tpu_sc as plsc

F32 = jnp.float32
I32 = jnp.int32

D = 256
HA = 16
B_EDGE = 100000
NTILE = 32
TK = 3200
BP = NTILE * TK
NBLK = TK // 128
R_ROWS = 4096
KB2 = 64
BM = 512


def _mesh():
    return plsc.VectorSubcoreMesh(core_axis_name="c", subcore_axis_name="s")


def _iota16():
    return jax.lax.iota(I32, 16)


def _mm_stacked(x, w_stack, bias, relu):
    n = x.shape[0]
    s_chunks = w_stack.shape[0]
    mb = pl.cdiv(n, BM)

    def body(x_ref, w_ref, b_ref, o_ref):
        acc = jnp.dot(x_ref[...], w_ref[0], preferred_element_type=F32)
        acc = acc + b_ref[0]
        if relu:
            acc = jnp.maximum(acc, 0.0)
        o_ref[0] = acc

    return pl.pallas_call(
        body,
        grid=(mb, s_chunks),
        in_specs=[
            pl.BlockSpec((BM, D), lambda i, j: (i, 0)),
            pl.BlockSpec((1, D, D), lambda i, j: (j, 0, 0)),
            pl.BlockSpec((1, 1, D), lambda i, j: (j, 0, 0)),
        ],
        out_specs=pl.BlockSpec((1, BM, D), lambda i, j: (j, i, 0)),
        out_shape=jax.ShapeDtypeStruct((s_chunks, n, D), F32),
    )(x, w_stack, bias[:, None, :])


def _mm_thin(x, wa):
    n = x.shape[0]
    na = wa.shape[1]
    mb = pl.cdiv(n, BM)

    def body(x_ref, w_ref, o_ref):
        o_ref[...] = jnp.dot(x_ref[...], w_ref[...], preferred_element_type=F32)

    return pl.pallas_call(
        body,
        grid=(mb,),
        in_specs=[
            pl.BlockSpec((BM, D), lambda i: (i, 0)),
            pl.BlockSpec((D, na), lambda i: (0, 0)),
        ],
        out_specs=pl.BlockSpec((BM, na), lambda i: (i, 0)),
        out_shape=jax.ShapeDtypeStruct((n, na), F32),
    )(x, wa)


def _combine2(a, b, bias, n):
    mb = pl.cdiv(n, BM)

    def body(a_ref, b_ref, bias_ref, o_ref):
        o_ref[...] = jnp.maximum(a_ref[...] + b_ref[...] + bias_ref[...], 0.0)

    return pl.pallas_call(
        body,
        grid=(mb,),
        in_specs=[
            pl.BlockSpec((BM, D), lambda i: (i, 0)),
            pl.BlockSpec((BM, D), lambda i: (i, 0)),
            pl.BlockSpec((1, D), lambda i: (0, 0)),
        ],
        out_specs=pl.BlockSpec((BM, D), lambda i: (i, 0)),
        out_shape=jax.ShapeDtypeStruct((n, D), F32),
    )(a, b, bias)


def _combine1(a, bias, n):
    mb = pl.cdiv(n, BM)

    def body(a_ref, bias_ref, o_ref):
        o_ref[...] = jnp.maximum(a_ref[...] + bias_ref[...], 0.0)

    return pl.pallas_call(
        body,
        grid=(mb,),
        in_specs=[
            pl.BlockSpec((BM, D), lambda i: (i, 0)),
            pl.BlockSpec((1, D), lambda i: (0, 0)),
        ],
        out_specs=pl.BlockSpec((BM, D), lambda i: (i, 0)),
        out_shape=jax.ShapeDtypeStruct((n, D), F32),
    )(a, bias)


@functools.partial(jax.jit, static_argnums=(4,))
def _sc_pass1(a_src_tab, a_dst_tab, src, dst, nd_pad):
    zfull, ztail = divmod(nd_pad // 16, 128)

    def body(asrc_hbm, adst_hbm, src_hbm, dst_hbm, ex_hbm, s_hbm,
             srcbuf, dstbuf, arow, brow, exbuf, zbuf, sem, s_acc):
        cid = lax.axis_index("c")
        sid = lax.axis_index("s")
        iota = _iota16()
        zero16 = jnp.zeros((16,), F32)

        def zb(j, _):
            zbuf[j] = zero16
            return 0
        lax.fori_loop(0, 128, zb, 0)

        rpt = nd_pad // 16
        def zs(j, _):
            pltpu.sync_copy(zbuf, s_acc.at[pl.ds(sid * rpt + j * 128, 128)])
            return 0
        lax.fori_loop(0, zfull, zs, 0)
        if ztail:
            pltpu.sync_copy(zbuf.at[pl.ds(0, ztail)],
                            s_acc.at[pl.ds(sid * rpt + zfull * 128, ztail)])
        plsc.subcore_barrier()

        tile_base = (cid * 16 + sid) * TK

        def blk(bi, _):
            base = tile_base + bi * 128
            pltpu.sync_copy(src_hbm.at[pl.ds(base, 128)], srcbuf)
            pltpu.sync_copy(dst_hbm.at[pl.ds(base, 128)], dstbuf)
            c1 = pltpu.async_copy(asrc_hbm.at[srcbuf], arow, sem)
            c2 = pltpu.async_copy(adst_hbm.at[dstbuf], brow, sem)
            c1.wait()
            c2.wait()

            def ew(j, _):
                e = arow[j] + brow[j]
                e = jnp.where(e > 0, e, 0.2 * e)
                exbuf[j] = jnp.exp(e)
                return 0
            lax.fori_loop(0, 128, ew, 0)

            pltpu.sync_copy(exbuf, ex_hbm.at[pl.ds(base, 128)])
            pltpu.sync_copy(exbuf, s_acc.at[dstbuf], add=True)
            return 0
        lax.fori_loop(0, NBLK, blk, 0)
        plsc.subcore_barrier()

        for t in range(zfull):
            pltpu.sync_copy(s_acc.at[pl.ds(sid * rpt + t * 128, 128)],
                            s_hbm.at[cid].at[pl.ds(sid * rpt + t * 128, 128)])
        if ztail:
            pltpu.sync_copy(s_acc.at[pl.ds(sid * rpt + zfull * 128, ztail)],
                            s_hbm.at[cid].at[pl.ds(sid * rpt + zfull * 128, ztail)])

    kern = pl.kernel(
        body,
        out_type=(
            jax.ShapeDtypeStruct((BP, HA), F32),
            jax.ShapeDtypeStruct((2, nd_pad, HA), F32),
        ),
        mesh=_mesh(),
        scratch_types=[
            pltpu.VMEM((128,), I32),
            pltpu.VMEM((128,), I32),
            pltpu.VMEM((128, HA), F32),
            pltpu.VMEM((128, HA), F32),
            pltpu.VMEM((128, HA), F32),
            pltpu.VMEM((128, HA), F32),
            pltpu.SemaphoreType.DMA,
            pltpu.VMEM_SHARED((nd_pad, HA), F32),
        ],
        compiler_params=pltpu.CompilerParams(use_tc_tiling_on_sc=False),
    )
    return kern(a_src_tab, a_dst_tab, src, dst)


@functools.partial(jax.jit, static_argnums=(5, 6, 7))
def _sc_pass2(h_stack, src, dst, ex, s_part, slot, nd_pad, chead):
    nranges = nd_pad // R_ROWS
    rpt2 = R_ROWS // 16
    tk2 = TK * 2

    def body(h_hbm, src_hbm, dst_hbm, ex_hbm, s_hbm, out_hbm,
             dstc, srcc, cb_src, cb_dst, cb_eid,
             gidx, sidx, eidx, scidx,
             rowbuf, exrow, s0row, s1row, wbuf, zbuf, sem, acc):
        cid = lax.axis_index("c")
        sid = lax.axis_index("s")
        iota = _iota16()
        tile_base = sid * tk2
        zero16 = jnp.zeros((16,), F32)

        pltpu.sync_copy(dst_hbm.at[pl.ds(tile_base, tk2)], dstc)
        pltpu.sync_copy(src_hbm.at[pl.ds(tile_base, tk2)], srcc)

        def zb(k, _):
            for j in range(16):
                zbuf[k, pl.ds(j * 16, 16)] = zero16
            return 0
        lax.fori_loop(0, 16, zb, 0)

        def range_body(r, _):
            lo = r * R_ROWS

            @pl.when(lax.rem(r, 2) == cid)
            def _():
                for t in range(rpt2 // 16):
                    pltpu.sync_copy(zbuf, acc.at[pl.ds(sid * rpt2 + t * 16, 16)])
                plsc.subcore_barrier()

                def scan(j, cnt_v):
                    d = dstc[pl.ds(j * 16, 16)]
                    s = srcc[pl.ds(j * 16, 16)]
                    m = (d >= lo) & (d < lo + R_ROWS)
                    pos = jnp.where(m, cnt_v + plsc.cumsum(m.astype(I32)) - 1,
                                    tk2 + 8)
                    plsc.store_scatter(cb_src, [pos], s)
                    plsc.store_scatter(cb_dst, [pos], d)
                    eid = (tile_base + j * 16) + iota
                    plsc.store_scatter(cb_eid, [pos], eid)
                    return cnt_v + plsc.all_reduce_population_count(m)
                cnt_v = lax.fori_loop(0, tk2 // 16, scan,
                                      jnp.zeros((16,), I32))
                cnt = cnt_v[0]

                nb = (cnt + (KB2 - 1)) // KB2

                def proc(bb, _):
                    k0 = bb * KB2

                    def mkidx(v, _):
                        pos = k0 + v * 16
                        m = (pos + iota) < cnt
                        sv = cb_src[pl.ds(pos, 16)]
                        dv = cb_dst[pl.ds(pos, 16)]
                        ev = cb_eid[pl.ds(pos, 16)]
                        gidx[pl.ds(v * 16, 16)] = jnp.where(m, sv, 0)
                        sidx[pl.ds(v * 16, 16)] = jnp.where(m, dv, 0)
                        eidx[pl.ds(v * 16, 16)] = jnp.where(m, ev, 0)
                        scidx[pl.ds(v * 16, 16)] = jnp.where(m, dv - lo, R_ROWS)
                        return 0
                    lax.fori_loop(0, KB2 // 16, mkidx, 0)

                    c1 = pltpu.async_copy(h_hbm.at[slot].at[gidx], rowbuf, sem)
                    c2 = pltpu.async_copy(ex_hbm.at[eidx], exrow, sem)
                    c3 = pltpu.async_copy(s_hbm.at[0].at[sidx], s0row, sem)
                    c4 = pltpu.async_copy(s_hbm.at[1].at[sidx], s1row, sem)
                    c1.wait()
                    c2.wait()
                    c3.wait()
                    c4.wait()

                    def ew(j, _):
                        wbuf[pl.ds(j * 16, 16)] = (
                            exrow[j] / (s0row[j] + s1row[j] + 1e-16))
                        return 0
                    lax.fori_loop(0, KB2, ew, 0)

                    def rowfn(e2, _):
                        for j in range(16):
                            hd = (16 * j) // chead
                            wsp = plsc.load_gather(
                                wbuf, [jnp.full((16,), e2 * 16 + hd, I32)])
                            rowbuf[e2, pl.ds(j * 16, 16)] = (
                                rowbuf[e2, pl.ds(j * 16, 16)] * wsp)
                        return 0
                    lax.fori_loop(0, KB2, rowfn, 0)

                    pltpu.sync_copy(rowbuf, acc.at[scidx], add=True)
                    return 0
                lax.fori_loop(0, nb, proc, 0)
                plsc.subcore_barrier()

                pltpu.sync_copy(acc.at[pl.ds(sid * rpt2, rpt2)],
                                out_hbm.at[pl.ds(lo + sid * rpt2, rpt2)])
            return 0
        lax.fori_loop(0, nranges, range_body, 0)

    kern = pl.kernel(
        body,
        out_type=jax.ShapeDtypeStruct((nd_pad, D), F32),
        mesh=_mesh(),
        scratch_types=[
            pltpu.VMEM((tk2,), I32),
            pltpu.VMEM((tk2,), I32),
            pltpu.VMEM((tk2 + 16,), I32),
            pltpu.VMEM((tk2 + 16,), I32),
            pltpu.VMEM((tk2 + 16,), I32),
            pltpu.VMEM((KB2,), I32),
            pltpu.VMEM((KB2,), I32),
            pltpu.VMEM((KB2,), I32),
            pltpu.VMEM((KB2,), I32),
            pltpu.VMEM((KB2, D), F32),
            pltpu.VMEM((KB2, HA), F32),
            pltpu.VMEM((KB2, HA), F32),
            pltpu.VMEM((KB2, HA), F32),
            pltpu.VMEM((KB2 * HA,), F32),
            pltpu.VMEM((16, D), F32),
            pltpu.SemaphoreType.DMA,
            pltpu.VMEM_SHARED((R_ROWS + 8, D), F32),
        ],
        compiler_params=pltpu.CompilerParams(use_tc_tiling_on_sc=False,
                                             needs_layout_passes=False),
    )
    return kern(h_stack, src, dst, ex, s_part)


def _att_fold(p, heads, chead):
    wr = p["W"].reshape(D, heads, chead)
    a_s = jnp.einsum("khc,hc->kh", wr, p["att_src"],
                     precision=jax.lax.Precision.HIGHEST)
    a_d = jnp.einsum("khc,hc->kh", wr, p["att_dst"],
                     precision=jax.lax.Precision.HIGHEST)
    if heads < HA:
        a_s = jnp.pad(a_s, ((0, 0), (0, HA - heads)))
        a_d = jnp.pad(a_d, ((0, 0), (0, HA - heads)))
    return a_s, a_d


def _pad_rows(a, extra=8):
    return jnp.pad(a, ((0, extra), (0, 0)))


def _pad_edges(e, n_dst):
    src = e[0].astype(I32)
    dst = e[1].astype(I32)
    pad = BP - B_EDGE
    src = jnp.concatenate([src, jnp.zeros((pad,), I32)])
    dst = jnp.concatenate([dst, jnp.full((pad,), n_dst, I32)])
    return src, dst


def kernel(x_individual, x_family, params,
           edge_index_individual_child_of_family,
           edge_index_family_parent_of_individual,
           edge_index_individual_spouse_individual):
    n_ind = x_individual.shape[0]
    n_fam = x_family.shape[0]
    ndp_ind = ((n_ind + 8 + R_ROWS - 1) // R_ROWS) * R_ROWS
    ndp_fam = ((n_fam + 8 + R_ROWS - 1) // R_ROWS) * R_ROWS

    s1e, d1e = _pad_edges(edge_index_individual_child_of_family, n_fam)
    s2e, d2e = _pad_edges(edge_index_family_parent_of_individual, n_ind)
    s3e, d3e = _pad_edges(edge_index_individual_spouse_individual, n_ind)

    pe_i = params["emb"]["individual"]
    pe_f = params["emb"]["family"]
    x_i = _mm_stacked(x_individual, pe_i["W"][None], pe_i["b"][None], True)[0]
    x_f = _mm_stacked(x_family, pe_f["W"][None], pe_f["b"][None], True)[0]

    k1 = "individual__child_of__family"
    k2 = "family__parent_of__individual"
    k3 = "individual__spouse__individual"

    for l in range(4):
        concat = l < 3
        heads = 8 if concat else 1
        chead = D // heads
        lp = params["convs"][l]
        p1, p2, p3 = lp[k1], lp[k2], lp[k3]

        u_ind = _mm_stacked(x_i, jnp.stack([p1["W"], p3["W"]]),
                            jnp.zeros((2, D), F32), False)
        u_fam = _mm_stacked(x_f, p2["W"][None], jnp.zeros((1, D), F32), False)

        a1s, a1d = _att_fold(p1, heads, chead)
        a2s, a2d = _att_fold(p2, heads, chead)
        a3s, a3d = _att_fold(p3, heads, chead)
        wa_ind = jnp.concatenate([a1s, a2d, a3s, a3d], axis=1)
        wa_fam = jnp.concatenate([a1d, a2s], axis=1)
        ai = _mm_thin(x_i, wa_ind)
        af = _mm_thin(x_f, wa_fam)

        t1s = _pad_rows(ai[:, 0:16])
        t2d = _pad_rows(ai[:, 16:32])
        t3s = _pad_rows(ai[:, 32:48])
        t3d = _pad_rows(ai[:, 48:64])
        t1d = _pad_rows(af[:, 0:16])
        t2s = _pad_rows(af[:, 16:32])

        ex1, sp1 = _sc_pass1(t1s, t1d, s1e, d1e, ndp_fam)
        ex2, sp2 = _sc_pass1(t2s, t2d, s2e, d2e, ndp_ind)
        ex3, sp3 = _sc_pass1(t3s, t3d, s3e, d3e, ndp_ind)

        o1 = _sc_pass2(u_ind, s1e, d1e, ex1, sp1, 0, ndp_fam, chead)
        o2 = _sc_pass2(u_fam, s2e, d2e, ex2, sp2, 0, ndp_ind, chead)
        o3 = _sc_pass2(u_ind, s3e, d3e, ex3, sp3, 1, ndp_ind, chead)

        x_f = _combine1(o1, p1["bias"][None], n_fam)
        x_i = _combine2(o2, o3, (p2["bias"] + p3["bias"])[None], n_ind)

    pf = params["pred"]["father"]
    pm = params["pred"]["mother"]
    pred = _mm_stacked(x_i, jnp.stack([pf["W"], pm["W"]]),
                       jnp.stack([pf["b"], pm["b"]]), False)
    return (x_i, x_f, pred[0], pred[1])


# speedup vs baseline: 14.0177x; 1.4841x over previous
import functools

import jax
import jax.numpy as jnp
from jax import lax
from jax.experimental import pallas as pl
from jax.experimental.pallas import tpu as pltpu
from jax.experimental.pallas import tpu_sc as plsc

F32 = jnp.float32
I32 = jnp.int32

D = 256
HA = 16
B_EDGE = 100000
NTILE = 32
TK = 3200
BP = NTILE * TK
NBLK = TK // 128
R_ROWS = 3584
KB2 = 64
BM = 512


def _mesh():
    return plsc.VectorSubcoreMesh(core_axis_name="c", subcore_axis_name="s")


def _iota16():
    return jax.lax.iota(I32, 16)


def _mm_stacked(x, w_stack, bias, relu):
    n = x.shape[0]
    s_chunks = w_stack.shape[0]
    mb = pl.cdiv(n, BM)

    def body(x_ref, w_ref, b_ref, o_ref):
        acc = jnp.dot(x_ref[...], w_ref[0], preferred_element_type=F32)
        acc = acc + b_ref[0]
        if relu:
            acc = jnp.maximum(acc, 0.0)
        o_ref[0] = acc

    return pl.pallas_call(
        body,
        grid=(mb, s_chunks),
        in_specs=[
            pl.BlockSpec((BM, D), lambda i, j: (i, 0)),
            pl.BlockSpec((1, D, D), lambda i, j: (j, 0, 0)),
            pl.BlockSpec((1, 1, D), lambda i, j: (j, 0, 0)),
        ],
        out_specs=pl.BlockSpec((1, BM, D), lambda i, j: (j, i, 0)),
        out_shape=jax.ShapeDtypeStruct((s_chunks, n, D), F32),
    )(x, w_stack, bias[:, None, :])


def _mm_thin(x, wa):
    n = x.shape[0]
    na = wa.shape[1]
    mb = pl.cdiv(n, BM)

    def body(x_ref, w_ref, o_ref):
        o_ref[...] = jnp.dot(x_ref[...], w_ref[...], preferred_element_type=F32)

    return pl.pallas_call(
        body,
        grid=(mb,),
        in_specs=[
            pl.BlockSpec((BM, D), lambda i: (i, 0)),
            pl.BlockSpec((D, na), lambda i: (0, 0)),
        ],
        out_specs=pl.BlockSpec((BM, na), lambda i: (i, 0)),
        out_shape=jax.ShapeDtypeStruct((n, na), F32),
    )(x, wa)


def _combine2(a, b, bias, n):
    mb = pl.cdiv(n, BM)

    def body(a_ref, b_ref, bias_ref, o_ref):
        o_ref[...] = jnp.maximum(a_ref[...] + b_ref[...] + bias_ref[...], 0.0)

    return pl.pallas_call(
        body,
        grid=(mb,),
        in_specs=[
            pl.BlockSpec((BM, D), lambda i: (i, 0)),
            pl.BlockSpec((BM, D), lambda i: (i, 0)),
            pl.BlockSpec((1, D), lambda i: (0, 0)),
        ],
        out_specs=pl.BlockSpec((BM, D), lambda i: (i, 0)),
        out_shape=jax.ShapeDtypeStruct((n, D), F32),
    )(a, b, bias)


def _combine1(a, bias, n):
    mb = pl.cdiv(n, BM)

    def body(a_ref, bias_ref, o_ref):
        o_ref[...] = jnp.maximum(a_ref[...] + bias_ref[...], 0.0)

    return pl.pallas_call(
        body,
        grid=(mb,),
        in_specs=[
            pl.BlockSpec((BM, D), lambda i: (i, 0)),
            pl.BlockSpec((1, D), lambda i: (0, 0)),
        ],
        out_specs=pl.BlockSpec((BM, D), lambda i: (i, 0)),
        out_shape=jax.ShapeDtypeStruct((n, D), F32),
    )(a, bias)


@functools.partial(jax.jit, static_argnums=(4,))
def _sc_pass1(a_src_tab, a_dst_tab, src, dst, nd_pad):
    zfull, ztail = divmod(nd_pad // 16, 128)

    def body(asrc_hbm, adst_hbm, src_hbm, dst_hbm, ex_hbm, s_hbm,
             srcbuf, dstbuf, arow, brow, exbuf, zbuf, sem, s_acc):
        cid = lax.axis_index("c")
        sid = lax.axis_index("s")
        iota = _iota16()
        zero16 = jnp.zeros((16,), F32)

        def zb(j, _):
            zbuf[j] = zero16
            return 0
        lax.fori_loop(0, 128, zb, 0)

        rpt = nd_pad // 16
        def zs(j, _):
            pltpu.sync_copy(zbuf, s_acc.at[pl.ds(sid * rpt + j * 128, 128)])
            return 0
        lax.fori_loop(0, zfull, zs, 0)
        if ztail:
            pltpu.sync_copy(zbuf.at[pl.ds(0, ztail)],
                            s_acc.at[pl.ds(sid * rpt + zfull * 128, ztail)])
        plsc.subcore_barrier()

        tile_base = (cid * 16 + sid) * TK

        def blk(bi, _):
            base = tile_base + bi * 128
            pltpu.sync_copy(src_hbm.at[pl.ds(base, 128)], srcbuf)
            pltpu.sync_copy(dst_hbm.at[pl.ds(base, 128)], dstbuf)
            c1 = pltpu.async_copy(asrc_hbm.at[srcbuf], arow, sem)
            c2 = pltpu.async_copy(adst_hbm.at[dstbuf], brow, sem)
            c1.wait()
            c2.wait()

            def ew(j, _):
                e = arow[j] + brow[j]
                e = jnp.where(e > 0, e, 0.2 * e)
                exbuf[j] = jnp.exp(e)
                return 0
            lax.fori_loop(0, 128, ew, 0)

            pltpu.sync_copy(exbuf, ex_hbm.at[pl.ds(base, 128)])
            pltpu.sync_copy(exbuf, s_acc.at[dstbuf], add=True)
            return 0
        lax.fori_loop(0, NBLK, blk, 0)
        plsc.subcore_barrier()

        for t in range(zfull):
            pltpu.sync_copy(s_acc.at[pl.ds(sid * rpt + t * 128, 128)],
                            s_hbm.at[cid].at[pl.ds(sid * rpt + t * 128, 128)])
        if ztail:
            pltpu.sync_copy(s_acc.at[pl.ds(sid * rpt + zfull * 128, ztail)],
                            s_hbm.at[cid].at[pl.ds(sid * rpt + zfull * 128, ztail)])

    kern = pl.kernel(
        body,
        out_type=(
            jax.ShapeDtypeStruct((BP, HA), F32),
            jax.ShapeDtypeStruct((2, nd_pad, HA), F32),
        ),
        mesh=_mesh(),
        scratch_types=[
            pltpu.VMEM((128,), I32),
            pltpu.VMEM((128,), I32),
            pltpu.VMEM((128, HA), F32),
            pltpu.VMEM((128, HA), F32),
            pltpu.VMEM((128, HA), F32),
            pltpu.VMEM((128, HA), F32),
            pltpu.SemaphoreType.DMA,
            pltpu.VMEM_SHARED((nd_pad, HA), F32),
        ],
        compiler_params=pltpu.CompilerParams(use_tc_tiling_on_sc=False),
    )
    return kern(a_src_tab, a_dst_tab, src, dst)


@functools.partial(jax.jit, static_argnums=(5, 6, 7))
def _sc_pass2(h_stack, src, dst, ex, s_part, hslot, nd_pad, chead):
    nranges = nd_pad // R_ROWS
    rpt2 = R_ROWS // 16
    tk2 = TK * 2

    def body(h_hbm, src_hbm, dst_hbm, ex_hbm, s_hbm, out_hbm,
             dstc, srcc, cb_src, cb_pk,
             gidxA, sidxA, eidxA, scidxA, gidxB, sidxB, eidxB, scidxB,
             rowA, exA, s0A, s1A, rowB, exB, s0B, s1B,
             wbuf, zbuf, semA, semB, acc):
        cid = lax.axis_index("c")
        sid = lax.axis_index("s")
        iota = _iota16()
        tile_base = sid * tk2
        zero16 = jnp.zeros((16,), F32)

        pltpu.sync_copy(dst_hbm.at[pl.ds(tile_base, tk2)], dstc)
        pltpu.sync_copy(src_hbm.at[pl.ds(tile_base, tk2)], srcc)

        def zb(k, _):
            for j in range(16):
                zbuf[k, pl.ds(j * 16, 16)] = zero16
            return 0
        lax.fori_loop(0, 8, zb, 0)

        def mk(cnt, k0, gidx, sidx, eidx, scidx, lo):
            def mkidx(v, _):
                pos = k0 + v * 16
                m = (pos + iota) < cnt
                sv = cb_src[pl.ds(pos, 16)]
                pk = cb_pk[pl.ds(pos, 16)]
                ev = pk & 0x1FFFF
                lv = lax.shift_right_logical(pk, 17)
                gidx[pl.ds(v * 16, 16)] = jnp.where(m, sv, 0)
                sidx[pl.ds(v * 16, 16)] = jnp.where(m, lv + lo, 0)
                eidx[pl.ds(v * 16, 16)] = jnp.where(m, ev, 0)
                scidx[pl.ds(v * 16, 16)] = jnp.where(m, lv, R_ROWS)
                return 0
            lax.fori_loop(0, KB2 // 16, mkidx, 0)

        def issue(gidx, sidx, eidx, rowb, exb, s0b, s1b, semx):
            pltpu.async_copy(h_hbm.at[hslot].at[gidx], rowb, semx)
            pltpu.async_copy(ex_hbm.at[eidx], exb, semx)
            pltpu.async_copy(s_hbm.at[0].at[sidx], s0b, semx)
            pltpu.async_copy(s_hbm.at[1].at[sidx], s1b, semx)

        def drain(gidx, sidx, eidx, rowb, exb, s0b, s1b, semx):
            pltpu.make_async_copy(h_hbm.at[hslot].at[gidx], rowb, semx).wait()
            pltpu.make_async_copy(ex_hbm.at[eidx], exb, semx).wait()
            pltpu.make_async_copy(s_hbm.at[0].at[sidx], s0b, semx).wait()
            pltpu.make_async_copy(s_hbm.at[1].at[sidx], s1b, semx).wait()

        def compute_scatter(rowb, exb, s0b, s1b, scidx):
            def ew(j, _):
                wbuf[pl.ds(j * 16, 16)] = (
                    exb[j] / (s0b[j] + s1b[j] + 1e-16))
                return 0
            lax.fori_loop(0, KB2, ew, 0)

            def rowfn(e2, _):
                wsp = None
                prev_hd = -1
                for j in range(16):
                    hd = (16 * j) // chead
                    if hd != prev_hd:
                        wsp = plsc.load_gather(
                            wbuf, [jnp.full((16,), e2 * 16 + hd, I32)])
                        prev_hd = hd
                    rowb[e2, pl.ds(j * 16, 16)] = (
                        rowb[e2, pl.ds(j * 16, 16)] * wsp)
                return 0
            lax.fori_loop(0, KB2, rowfn, 0)
            pltpu.sync_copy(rowb, acc.at[scidx], add=True)

        def range_body(r, _):
            lo = r * R_ROWS

            @pl.when(lax.rem(r, 2) == cid)
            def _():
                for t in range(rpt2 // 8):
                    pltpu.sync_copy(zbuf, acc.at[pl.ds(sid * rpt2 + t * 8, 8)])
                plsc.subcore_barrier()

                def scan(j, cnt_v):
                    d = dstc[pl.ds(j * 16, 16)]
                    s = srcc[pl.ds(j * 16, 16)]
                    lv = d - lo
                    m = (lv >= 0) & (lv < R_ROWS)
                    pos = jnp.where(m, cnt_v + plsc.cumsum(m.astype(I32)) - 1,
                                    tk2 + 8)
                    plsc.store_scatter(cb_src, [pos], s)
                    eid = (tile_base + j * 16) + iota
                    plsc.store_scatter(cb_pk, [pos],
                                       eid | lax.shift_left(lv, 17))
                    return cnt_v + plsc.all_reduce_population_count(m)
                cnt_v = lax.fori_loop(0, tk2 // 16, scan,
                                      jnp.zeros((16,), I32))
                cnt = cnt_v[0]

                nb = (cnt + (KB2 - 1)) // KB2

                @pl.when(nb > 0)
                def _():
                    mk(cnt, 0, gidxA, sidxA, eidxA, scidxA, lo)
                    issue(gidxA, sidxA, eidxA, rowA, exA, s0A, s1A, semA)

                def proc2(t, _):
                    bb = t * 2

                    drain(gidxA, sidxA, eidxA, rowA, exA, s0A, s1A, semA)

                    @pl.when(bb + 1 < nb)
                    def _():
                        mk(cnt, (bb + 1) * KB2, gidxB, sidxB, eidxB, scidxB,
                           lo)
                        issue(gidxB, sidxB, eidxB, rowB, exB, s0B, s1B, semB)

                    compute_scatter(rowA, exA, s0A, s1A, scidxA)

                    @pl.when(bb + 1 < nb)
                    def _():
                        drain(gidxB, sidxB, eidxB, rowB, exB, s0B, s1B, semB)

                        @pl.when(bb + 2 < nb)
                        def _():
                            mk(cnt, (bb + 2) * KB2, gidxA, sidxA, eidxA,
                               scidxA, lo)
                            issue(gidxA, sidxA, eidxA, rowA, exA, s0A, s1A,
                                  semA)

                        compute_scatter(rowB, exB, s0B, s1B, scidxB)
                    return 0
                lax.fori_loop(0, (nb + 1) // 2, proc2, 0)
                plsc.subcore_barrier()

                pltpu.sync_copy(acc.at[pl.ds(sid * rpt2, rpt2)],
                                out_hbm.at[pl.ds(lo + sid * rpt2, rpt2)])
            return 0
        lax.fori_loop(0, nranges, range_body, 0)

    kern = pl.kernel(
        body,
        out_type=jax.ShapeDtypeStruct((nd_pad, D), F32),
        mesh=_mesh(),
        scratch_types=[
            pltpu.VMEM((tk2,), I32),
            pltpu.VMEM((tk2,), I32),
            pltpu.VMEM((tk2 + 16,), I32),
            pltpu.VMEM((tk2 + 16,), I32),
            pltpu.VMEM((KB2,), I32),
            pltpu.VMEM((KB2,), I32),
            pltpu.VMEM((KB2,), I32),
            pltpu.VMEM((KB2,), I32),
            pltpu.VMEM((KB2,), I32),
            pltpu.VMEM((KB2,), I32),
            pltpu.VMEM((KB2,), I32),
            pltpu.VMEM((KB2,), I32),
            pltpu.VMEM((KB2, D), F32),
            pltpu.VMEM((KB2, HA), F32),
            pltpu.VMEM((KB2, HA), F32),
            pltpu.VMEM((KB2, HA), F32),
            pltpu.VMEM((KB2, D), F32),
            pltpu.VMEM((KB2, HA), F32),
            pltpu.VMEM((KB2, HA), F32),
            pltpu.VMEM((KB2, HA), F32),
            pltpu.VMEM((KB2 * HA,), F32),
            pltpu.VMEM((8, D), F32),
            pltpu.SemaphoreType.DMA,
            pltpu.SemaphoreType.DMA,
            pltpu.VMEM_SHARED((R_ROWS + 8, D), F32),
        ],
        compiler_params=pltpu.CompilerParams(use_tc_tiling_on_sc=False,
                                             needs_layout_passes=False),
    )
    return kern(h_stack, src, dst, ex, s_part)


def _att_fold(p, heads, chead):
    wr = p["W"].reshape(D, heads, chead)
    a_s = jnp.einsum("khc,hc->kh", wr, p["att_src"],
                     precision=jax.lax.Precision.HIGHEST)
    a_d = jnp.einsum("khc,hc->kh", wr, p["att_dst"],
                     precision=jax.lax.Precision.HIGHEST)
    if heads < HA:
        a_s = jnp.pad(a_s, ((0, 0), (0, HA - heads)))
        a_d = jnp.pad(a_d, ((0, 0), (0, HA - heads)))
    return a_s, a_d


def _pad_rows(a, extra=8):
    return jnp.pad(a, ((0, extra), (0, 0)))


def _pad_edges(e, n_dst):
    src = e[0].astype(I32)
    dst = e[1].astype(I32)
    pad = BP - B_EDGE
    src = jnp.concatenate([src, jnp.zeros((pad,), I32)])
    dst = jnp.concatenate([dst, jnp.full((pad,), n_dst, I32)])
    return src, dst


def kernel(x_individual, x_family, params,
           edge_index_individual_child_of_family,
           edge_index_family_parent_of_individual,
           edge_index_individual_spouse_individual):
    n_ind = x_individual.shape[0]
    n_fam = x_family.shape[0]
    ndp_ind = ((n_ind + 8 + R_ROWS - 1) // R_ROWS) * R_ROWS
    ndp_fam = ((n_fam + 8 + R_ROWS - 1) // R_ROWS) * R_ROWS

    s1e, d1e = _pad_edges(edge_index_individual_child_of_family, n_fam)
    s2e, d2e = _pad_edges(edge_index_family_parent_of_individual, n_ind)
    s3e, d3e = _pad_edges(edge_index_individual_spouse_individual, n_ind)

    pe_i = params["emb"]["individual"]
    pe_f = params["emb"]["family"]
    x_i = _mm_stacked(x_individual, pe_i["W"][None], pe_i["b"][None], True)[0]
    x_f = _mm_stacked(x_family, pe_f["W"][None], pe_f["b"][None], True)[0]

    k1 = "individual__child_of__family"
    k2 = "family__parent_of__individual"
    k3 = "individual__spouse__individual"

    for l in range(4):
        concat = l < 3
        heads = 8 if concat else 1
        chead = D // heads
        lp = params["convs"][l]
        p1, p2, p3 = lp[k1], lp[k2], lp[k3]

        u_ind = _mm_stacked(x_i, jnp.stack([p1["W"], p3["W"]]),
                            jnp.zeros((2, D), F32), False)
        u_fam = _mm_stacked(x_f, p2["W"][None], jnp.zeros((1, D), F32), False)

        a1s, a1d = _att_fold(p1, heads, chead)
        a2s, a2d = _att_fold(p2, heads, chead)
        a3s, a3d = _att_fold(p3, heads, chead)
        wa_ind = jnp.concatenate([a1s, a2d, a3s, a3d], axis=1)
        wa_fam = jnp.concatenate([a1d, a2s], axis=1)
        ai = _mm_thin(x_i, wa_ind)
        af = _mm_thin(x_f, wa_fam)

        t1s = _pad_rows(ai[:, 0:16])
        t2d = _pad_rows(ai[:, 16:32])
        t3s = _pad_rows(ai[:, 32:48])
        t3d = _pad_rows(ai[:, 48:64])
        t1d = _pad_rows(af[:, 0:16])
        t2s = _pad_rows(af[:, 16:32])

        ex1, sp1 = _sc_pass1(t1s, t1d, s1e, d1e, ndp_fam)
        ex2, sp2 = _sc_pass1(t2s, t2d, s2e, d2e, ndp_ind)
        ex3, sp3 = _sc_pass1(t3s, t3d, s3e, d3e, ndp_ind)

        o1 = _sc_pass2(u_ind, s1e, d1e, ex1, sp1, 0, ndp_fam, chead)
        o2 = _sc_pass2(u_fam, s2e, d2e, ex2, sp2, 0, ndp_ind, chead)
        o3 = _sc_pass2(u_ind, s3e, d3e, ex3, sp3, 1, ndp_ind, chead)

        x_f = _combine1(o1, p1["bias"][None], n_fam)
        x_i = _combine2(o2, o3, (p2["bias"] + p3["bias"])[None], n_ind)

    pf = params["pred"]["father"]
    pm = params["pred"]["mother"]
    pred = _mm_stacked(x_i, jnp.stack([pf["W"], pm["W"]]),
                       jnp.stack([pf["b"], pm["b"]]), False)
    return (x_i, x_f, pred[0], pred[1])


# speedup vs baseline: 14.3523x; 1.0239x over previous
import functools

import jax
import jax.numpy as jnp
from jax import lax
from jax.experimental import pallas as pl
from jax.experimental.pallas import tpu as pltpu
from jax.experimental.pallas import tpu_sc as plsc

F32 = jnp.float32
I32 = jnp.int32

D = 256
HA = 16
B_EDGE = 100000
NTILE = 32
TK = 3200
BP = NTILE * TK
NBLK = TK // 128
R_ROWS = 3584
KB2 = 64
BM = 512


def _mesh():
    return plsc.VectorSubcoreMesh(core_axis_name="c", subcore_axis_name="s")


def _iota16():
    return jax.lax.iota(I32, 16)


def _mm_stacked(x, w_stack, bias, relu):
    n = x.shape[0]
    s_chunks = w_stack.shape[0]
    mb = pl.cdiv(n, BM)

    def body(x_ref, w_ref, b_ref, o_ref):
        acc = jnp.dot(x_ref[...], w_ref[0], preferred_element_type=F32)
        acc = acc + b_ref[0]
        if relu:
            acc = jnp.maximum(acc, 0.0)
        o_ref[0] = acc

    return pl.pallas_call(
        body,
        grid=(mb, s_chunks),
        in_specs=[
            pl.BlockSpec((BM, D), lambda i, j: (i, 0)),
            pl.BlockSpec((1, D, D), lambda i, j: (j, 0, 0)),
            pl.BlockSpec((1, 1, D), lambda i, j: (j, 0, 0)),
        ],
        out_specs=pl.BlockSpec((1, BM, D), lambda i, j: (j, i, 0)),
        out_shape=jax.ShapeDtypeStruct((s_chunks, n, D), F32),
    )(x, w_stack, bias[:, None, :])


def _mm_thin(x, wa):
    n = x.shape[0]
    na = wa.shape[1]
    mb = pl.cdiv(n, BM)

    def body(x_ref, w_ref, o_ref):
        o_ref[...] = jnp.dot(x_ref[...], w_ref[...], preferred_element_type=F32)

    return pl.pallas_call(
        body,
        grid=(mb,),
        in_specs=[
            pl.BlockSpec((BM, D), lambda i: (i, 0)),
            pl.BlockSpec((D, na), lambda i: (0, 0)),
        ],
        out_specs=pl.BlockSpec((BM, na), lambda i: (i, 0)),
        out_shape=jax.ShapeDtypeStruct((n, na), F32),
    )(x, wa)


def _combine2(a, b, bias, n):
    mb = pl.cdiv(n, BM)

    def body(a_ref, b_ref, bias_ref, o_ref):
        o_ref[...] = jnp.maximum(a_ref[...] + b_ref[...] + bias_ref[...], 0.0)

    return pl.pallas_call(
        body,
        grid=(mb,),
        in_specs=[
            pl.BlockSpec((BM, D), lambda i: (i, 0)),
            pl.BlockSpec((BM, D), lambda i: (i, 0)),
            pl.BlockSpec((1, D), lambda i: (0, 0)),
        ],
        out_specs=pl.BlockSpec((BM, D), lambda i: (i, 0)),
        out_shape=jax.ShapeDtypeStruct((n, D), F32),
    )(a, b, bias)


def _combine1(a, bias, n):
    mb = pl.cdiv(n, BM)

    def body(a_ref, bias_ref, o_ref):
        o_ref[...] = jnp.maximum(a_ref[...] + bias_ref[...], 0.0)

    return pl.pallas_call(
        body,
        grid=(mb,),
        in_specs=[
            pl.BlockSpec((BM, D), lambda i: (i, 0)),
            pl.BlockSpec((1, D), lambda i: (0, 0)),
        ],
        out_specs=pl.BlockSpec((BM, D), lambda i: (i, 0)),
        out_shape=jax.ShapeDtypeStruct((n, D), F32),
    )(a, bias)


@functools.partial(jax.jit, static_argnums=(4,))
def _sc_pass1(a_src_tab, a_dst_tab, src, dst, nd_pad):
    zfull, ztail = divmod(nd_pad // 16, 128)

    def body(asrc_hbm, adst_hbm, src_hbm, dst_hbm, ex_hbm, s_hbm,
             srcc, dstc, arowA, browA, arowB, browB, exbuf, zbuf, dstbuf,
             semA, semB, s_acc):
        cid = lax.axis_index("c")
        sid = lax.axis_index("s")
        zero16 = jnp.zeros((16,), F32)

        tile_base = (cid * 16 + sid) * TK

        pltpu.sync_copy(src_hbm.at[pl.ds(tile_base, TK)], srcc)
        pltpu.sync_copy(dst_hbm.at[pl.ds(tile_base, TK)], dstc)

        def zb(j, _):
            zbuf[j] = zero16
            return 0
        lax.fori_loop(0, 128, zb, 0)

        rpt = nd_pad // 16
        def zs(j, _):
            pltpu.sync_copy(zbuf, s_acc.at[pl.ds(sid * rpt + j * 128, 128)])
            return 0
        lax.fori_loop(0, zfull, zs, 0)
        if ztail:
            pltpu.sync_copy(zbuf.at[pl.ds(0, ztail)],
                            s_acc.at[pl.ds(sid * rpt + zfull * 128, ztail)])
        plsc.subcore_barrier()

        def issue(bi, arow, brow, semx):
            pltpu.async_copy(asrc_hbm.at[srcc.at[pl.ds(bi * 128, 128)]],
                             arow, semx)
            pltpu.async_copy(adst_hbm.at[dstc.at[pl.ds(bi * 128, 128)]],
                             brow, semx)

        def drain(bi, arow, brow, semx):
            pltpu.make_async_copy(asrc_hbm.at[srcc.at[pl.ds(bi * 128, 128)]],
                                  arow, semx).wait()
            pltpu.make_async_copy(adst_hbm.at[dstc.at[pl.ds(bi * 128, 128)]],
                                  brow, semx).wait()

        def compute(bi, arow, brow):
            base = tile_base + bi * 128

            def ew(j, _):
                e = arow[j] + brow[j]
                e = jnp.where(e > 0, e, 0.2 * e)
                exbuf[j] = jnp.exp(e)
                return 0
            lax.fori_loop(0, 128, ew, 0)

            pltpu.sync_copy(exbuf, ex_hbm.at[pl.ds(base, 128)])
            def cpi(v, _):
                dstbuf[pl.ds(v * 16, 16)] = dstc[pl.ds(bi * 128 + v * 16, 16)]
                return 0
            lax.fori_loop(0, 8, cpi, 0)
            pltpu.sync_copy(exbuf, s_acc.at[dstbuf], add=True)

        issue(0, arowA, browA, semA)

        def blk2(t, _):
            bi = t * 2
            drain(bi, arowA, browA, semA)

            @pl.when(bi + 1 < NBLK)
            def _():
                issue(bi + 1, arowB, browB, semB)
            compute(bi, arowA, browA)

            @pl.when(bi + 1 < NBLK)
            def _():
                drain(bi + 1, arowB, browB, semB)

                @pl.when(bi + 2 < NBLK)
                def _():
                    issue(bi + 2, arowA, browA, semA)
                compute(bi + 1, arowB, browB)
            return 0
        lax.fori_loop(0, (NBLK + 1) // 2, blk2, 0)
        plsc.subcore_barrier()

        for t in range(zfull):
            pltpu.sync_copy(s_acc.at[pl.ds(sid * rpt + t * 128, 128)],
                            s_hbm.at[cid].at[pl.ds(sid * rpt + t * 128, 128)])
        if ztail:
            pltpu.sync_copy(s_acc.at[pl.ds(sid * rpt + zfull * 128, ztail)],
                            s_hbm.at[cid].at[pl.ds(sid * rpt + zfull * 128, ztail)])

    kern = pl.kernel(
        body,
        out_type=(
            jax.ShapeDtypeStruct((BP, HA), F32),
            jax.ShapeDtypeStruct((2, nd_pad, HA), F32),
        ),
        mesh=_mesh(),
        scratch_types=[
            pltpu.VMEM((TK,), I32),
            pltpu.VMEM((TK,), I32),
            pltpu.VMEM((128, HA), F32),
            pltpu.VMEM((128, HA), F32),
            pltpu.VMEM((128, HA), F32),
            pltpu.VMEM((128, HA), F32),
            pltpu.VMEM((128, HA), F32),
            pltpu.VMEM((128, HA), F32),
            pltpu.VMEM((128,), I32),
            pltpu.SemaphoreType.DMA,
            pltpu.SemaphoreType.DMA,
            pltpu.VMEM_SHARED((nd_pad, HA), F32),
        ],
        compiler_params=pltpu.CompilerParams(use_tc_tiling_on_sc=False),
    )
    return kern(a_src_tab, a_dst_tab, src, dst)


@functools.partial(jax.jit, static_argnums=(5, 6, 7))
def _sc_pass2(h_stack, src, dst, ex, s_part, hslot, nd_pad, chead):
    nranges = nd_pad // R_ROWS
    rpt2 = R_ROWS // 16
    tk2 = TK * 2

    def body(h_hbm, src_hbm, dst_hbm, ex_hbm, s_hbm, out_hbm,
             dstc, srcc, cb_src, cb_pk,
             gidxA, sidxA, eidxA, scidxA, gidxB, sidxB, eidxB, scidxB,
             rowA, exA, s0A, s1A, rowB, exB, s0B, s1B,
             wbuf, zbuf, semA, semB, acc):
        cid = lax.axis_index("c")
        sid = lax.axis_index("s")
        iota = _iota16()
        tile_base = sid * tk2
        zero16 = jnp.zeros((16,), F32)

        pltpu.sync_copy(dst_hbm.at[pl.ds(tile_base, tk2)], dstc)
        pltpu.sync_copy(src_hbm.at[pl.ds(tile_base, tk2)], srcc)

        def zb(k, _):
            for j in range(16):
                zbuf[k, pl.ds(j * 16, 16)] = zero16
            return 0
        lax.fori_loop(0, 8, zb, 0)

        def mk(cnt, k0, gidx, sidx, eidx, scidx, lo):
            def mkidx(v, _):
                pos = k0 + v * 16
                m = (pos + iota) < cnt
                sv = cb_src[pl.ds(pos, 16)]
                pk = cb_pk[pl.ds(pos, 16)]
                ev = pk & 0x1FFFF
                lv = lax.shift_right_logical(pk, 17)
                gidx[pl.ds(v * 16, 16)] = jnp.where(m, sv, 0)
                sidx[pl.ds(v * 16, 16)] = jnp.where(m, lv + lo, 0)
                eidx[pl.ds(v * 16, 16)] = jnp.where(m, ev, 0)
                scidx[pl.ds(v * 16, 16)] = jnp.where(m, lv, R_ROWS)
                return 0
            lax.fori_loop(0, KB2 // 16, mkidx, 0)

        def issue(gidx, sidx, eidx, rowb, exb, s0b, s1b, semx):
            pltpu.async_copy(h_hbm.at[hslot].at[gidx], rowb, semx)
            pltpu.async_copy(ex_hbm.at[eidx], exb, semx)
            pltpu.async_copy(s_hbm.at[0].at[sidx], s0b, semx)
            pltpu.async_copy(s_hbm.at[1].at[sidx], s1b, semx)

        def drain(gidx, sidx, eidx, rowb, exb, s0b, s1b, semx):
            pltpu.make_async_copy(h_hbm.at[hslot].at[gidx], rowb, semx).wait()
            pltpu.make_async_copy(ex_hbm.at[eidx], exb, semx).wait()
            pltpu.make_async_copy(s_hbm.at[0].at[sidx], s0b, semx).wait()
            pltpu.make_async_copy(s_hbm.at[1].at[sidx], s1b, semx).wait()

        def compute_scatter(rowb, exb, s0b, s1b, scidx):
            def ew(j, _):
                wbuf[pl.ds(j * 16, 16)] = (
                    exb[j] / (s0b[j] + s1b[j] + 1e-16))
                return 0
            lax.fori_loop(0, KB2, ew, 0)

            def rowfn(e2, _):
                wsp = None
                prev_hd = -1
                for j in range(16):
                    hd = (16 * j) // chead
                    if hd != prev_hd:
                        wsp = plsc.load_gather(
                            wbuf, [jnp.full((16,), e2 * 16 + hd, I32)])
                        prev_hd = hd
                    rowb[e2, pl.ds(j * 16, 16)] = (
                        rowb[e2, pl.ds(j * 16, 16)] * wsp)
                return 0
            lax.fori_loop(0, KB2, rowfn, 0)
            pltpu.sync_copy(rowb, acc.at[scidx], add=True)

        def range_body(r, _):
            lo = r * R_ROWS

            @pl.when(lax.rem(r, 2) == cid)
            def _():
                for t in range(rpt2 // 8):
                    pltpu.sync_copy(zbuf, acc.at[pl.ds(sid * rpt2 + t * 8, 8)])
                plsc.subcore_barrier()

                def scan(j, cnt_v):
                    d = dstc[pl.ds(j * 16, 16)]
                    s = srcc[pl.ds(j * 16, 16)]
                    lv = d - lo
                    m = (lv >= 0) & (lv < R_ROWS)
                    pos = jnp.where(m, cnt_v + plsc.cumsum(m.astype(I32)) - 1,
                                    tk2 + 8)
                    plsc.store_scatter(cb_src, [pos], s)
                    eid = (tile_base + j * 16) + iota
                    plsc.store_scatter(cb_pk, [pos],
                                       eid | lax.shift_left(lv, 17))
                    return cnt_v + plsc.all_reduce_population_count(m)
                cnt_v = lax.fori_loop(0, tk2 // 16, scan,
                                      jnp.zeros((16,), I32))
                cnt = cnt_v[0]

                nb = (cnt + (KB2 - 1)) // KB2

                @pl.when(nb > 0)
                def _():
                    mk(cnt, 0, gidxA, sidxA, eidxA, scidxA, lo)
                    issue(gidxA, sidxA, eidxA, rowA, exA, s0A, s1A, semA)

                def proc2(t, _):
                    bb = t * 2

                    drain(gidxA, sidxA, eidxA, rowA, exA, s0A, s1A, semA)

                    @pl.when(bb + 1 < nb)
                    def _():
                        mk(cnt, (bb + 1) * KB2, gidxB, sidxB, eidxB, scidxB,
                           lo)
                        issue(gidxB, sidxB, eidxB, rowB, exB, s0B, s1B, semB)

                    compute_scatter(rowA, exA, s0A, s1A, scidxA)

                    @pl.when(bb + 1 < nb)
                    def _():
                        drain(gidxB, sidxB, eidxB, rowB, exB, s0B, s1B, semB)

                        @pl.when(bb + 2 < nb)
                        def _():
                            mk(cnt, (bb + 2) * KB2, gidxA, sidxA, eidxA,
                               scidxA, lo)
                            issue(gidxA, sidxA, eidxA, rowA, exA, s0A, s1A,
                                  semA)

                        compute_scatter(rowB, exB, s0B, s1B, scidxB)
                    return 0
                lax.fori_loop(0, (nb + 1) // 2, proc2, 0)
                plsc.subcore_barrier()

                pltpu.sync_copy(acc.at[pl.ds(sid * rpt2, rpt2)],
                                out_hbm.at[pl.ds(lo + sid * rpt2, rpt2)])
            return 0
        lax.fori_loop(0, nranges, range_body, 0)

    kern = pl.kernel(
        body,
        out_type=jax.ShapeDtypeStruct((nd_pad, D), F32),
        mesh=_mesh(),
        scratch_types=[
            pltpu.VMEM((tk2,), I32),
            pltpu.VMEM((tk2,), I32),
            pltpu.VMEM((tk2 + 16,), I32),
            pltpu.VMEM((tk2 + 16,), I32),
            pltpu.VMEM((KB2,), I32),
            pltpu.VMEM((KB2,), I32),
            pltpu.VMEM((KB2,), I32),
            pltpu.VMEM((KB2,), I32),
            pltpu.VMEM((KB2,), I32),
            pltpu.VMEM((KB2,), I32),
            pltpu.VMEM((KB2,), I32),
            pltpu.VMEM((KB2,), I32),
            pltpu.VMEM((KB2, D), F32),
            pltpu.VMEM((KB2, HA), F32),
            pltpu.VMEM((KB2, HA), F32),
            pltpu.VMEM((KB2, HA), F32),
            pltpu.VMEM((KB2, D), F32),
            pltpu.VMEM((KB2, HA), F32),
            pltpu.VMEM((KB2, HA), F32),
            pltpu.VMEM((KB2, HA), F32),
            pltpu.VMEM((KB2 * HA,), F32),
            pltpu.VMEM((8, D), F32),
            pltpu.SemaphoreType.DMA,
            pltpu.SemaphoreType.DMA,
            pltpu.VMEM_SHARED((R_ROWS + 8, D), F32),
        ],
        compiler_params=pltpu.CompilerParams(use_tc_tiling_on_sc=False,
                                             needs_layout_passes=False),
    )
    return kern(h_stack, src, dst, ex, s_part)


def _att_fold(p, heads, chead):
    wr = p["W"].reshape(D, heads, chead)
    a_s = jnp.einsum("khc,hc->kh", wr, p["att_src"],
                     precision=jax.lax.Precision.HIGHEST)
    a_d = jnp.einsum("khc,hc->kh", wr, p["att_dst"],
                     precision=jax.lax.Precision.HIGHEST)
    if heads < HA:
        a_s = jnp.pad(a_s, ((0, 0), (0, HA - heads)))
        a_d = jnp.pad(a_d, ((0, 0), (0, HA - heads)))
    return a_s, a_d


def _pad_rows(a, extra=8):
    return jnp.pad(a, ((0, extra), (0, 0)))


def _pad_edges(e, n_dst):
    src = e[0].astype(I32)
    dst = e[1].astype(I32)
    pad = BP - B_EDGE
    src = jnp.concatenate([src, jnp.zeros((pad,), I32)])
    dst = jnp.concatenate([dst, jnp.full((pad,), n_dst, I32)])
    return src, dst


def kernel(x_individual, x_family, params,
           edge_index_individual_child_of_family,
           edge_index_family_parent_of_individual,
           edge_index_individual_spouse_individual):
    n_ind = x_individual.shape[0]
    n_fam = x_family.shape[0]
    ndp_ind = ((n_ind + 8 + R_ROWS - 1) // R_ROWS) * R_ROWS
    ndp_fam = ((n_fam + 8 + R_ROWS - 1) // R_ROWS) * R_ROWS

    s1e, d1e = _pad_edges(edge_index_individual_child_of_family, n_fam)
    s2e, d2e = _pad_edges(edge_index_family_parent_of_individual, n_ind)
    s3e, d3e = _pad_edges(edge_index_individual_spouse_individual, n_ind)

    pe_i = params["emb"]["individual"]
    pe_f = params["emb"]["family"]
    x_i = _mm_stacked(x_individual, pe_i["W"][None], pe_i["b"][None], True)[0]
    x_f = _mm_stacked(x_family, pe_f["W"][None], pe_f["b"][None], True)[0]

    k1 = "individual__child_of__family"
    k2 = "family__parent_of__individual"
    k3 = "individual__spouse__individual"

    for l in range(4):
        concat = l < 3
        heads = 8 if concat else 1
        chead = D // heads
        lp = params["convs"][l]
        p1, p2, p3 = lp[k1], lp[k2], lp[k3]

        u_ind = _mm_stacked(x_i, jnp.stack([p1["W"], p3["W"]]),
                            jnp.zeros((2, D), F32), False)
        u_fam = _mm_stacked(x_f, p2["W"][None], jnp.zeros((1, D), F32), False)

        a1s, a1d = _att_fold(p1, heads, chead)
        a2s, a2d = _att_fold(p2, heads, chead)
        a3s, a3d = _att_fold(p3, heads, chead)
        wa_ind = jnp.concatenate([a1s, a2d, a3s, a3d], axis=1)
        wa_fam = jnp.concatenate([a1d, a2s], axis=1)
        ai = _mm_thin(x_i, wa_ind)
        af = _mm_thin(x_f, wa_fam)

        t1s = _pad_rows(ai[:, 0:16])
        t2d = _pad_rows(ai[:, 16:32])
        t3s = _pad_rows(ai[:, 32:48])
        t3d = _pad_rows(ai[:, 48:64])
        t1d = _pad_rows(af[:, 0:16])
        t2s = _pad_rows(af[:, 16:32])

        ex1, sp1 = _sc_pass1(t1s, t1d, s1e, d1e, ndp_fam)
        ex2, sp2 = _sc_pass1(t2s, t2d, s2e, d2e, ndp_ind)
        ex3, sp3 = _sc_pass1(t3s, t3d, s3e, d3e, ndp_ind)

        o1 = _sc_pass2(u_ind, s1e, d1e, ex1, sp1, 0, ndp_fam, chead)
        o2 = _sc_pass2(u_fam, s2e, d2e, ex2, sp2, 0, ndp_ind, chead)
        o3 = _sc_pass2(u_ind, s3e, d3e, ex3, sp3, 1, ndp_ind, chead)

        x_f = _combine1(o1, p1["bias"][None], n_fam)
        x_i = _combine2(o2, o3, (p2["bias"] + p3["bias"])[None], n_ind)

    pf = params["pred"]["father"]
    pm = params["pred"]["mother"]
    pred = _mm_stacked(x_i, jnp.stack([pf["W"], pm["W"]]),
                       jnp.stack([pf["b"], pm["b"]]), False)
    return (x_i, x_f, pred[0], pred[1])


# speedup vs baseline: 14.5735x; 1.0154x over previous
import functools

import jax
import jax.numpy as jnp
from jax import lax
from jax.experimental import pallas as pl
from jax.experimental.pallas import tpu as pltpu
from jax.experimental.pallas import tpu_sc as plsc

F32 = jnp.float32
I32 = jnp.int32

D = 256
HA = 16
B_EDGE = 100000
NTILE = 32
TK = 3200
BP = NTILE * TK
NBLK = TK // 128
R_ROWS = 3584
KB2 = 64
BM = 512


def _mesh():
    return plsc.VectorSubcoreMesh(core_axis_name="c", subcore_axis_name="s")


def _iota16():
    return jax.lax.iota(I32, 16)


def _mm_stacked(x, w_stack, bias, relu):
    n = x.shape[0]
    s_chunks = w_stack.shape[0]
    mb = pl.cdiv(n, BM)

    def body(x_ref, w_ref, b_ref, o_ref):
        acc = jnp.dot(x_ref[...], w_ref[0], preferred_element_type=F32)
        acc = acc + b_ref[0]
        if relu:
            acc = jnp.maximum(acc, 0.0)
        o_ref[0] = acc

    return pl.pallas_call(
        body,
        grid=(mb, s_chunks),
        in_specs=[
            pl.BlockSpec((BM, D), lambda i, j: (i, 0)),
            pl.BlockSpec((1, D, D), lambda i, j: (j, 0, 0)),
            pl.BlockSpec((1, 1, D), lambda i, j: (j, 0, 0)),
        ],
        out_specs=pl.BlockSpec((1, BM, D), lambda i, j: (j, i, 0)),
        out_shape=jax.ShapeDtypeStruct((s_chunks, n, D), F32),
    )(x, w_stack, bias[:, None, :])


def _mm_thin(x, wa):
    n = x.shape[0]
    na = wa.shape[1]
    mb = pl.cdiv(n, BM)

    def body(x_ref, w_ref, o_ref):
        o_ref[...] = jnp.dot(x_ref[...], w_ref[...], preferred_element_type=F32)

    return pl.pallas_call(
        body,
        grid=(mb,),
        in_specs=[
            pl.BlockSpec((BM, D), lambda i: (i, 0)),
            pl.BlockSpec((D, na), lambda i: (0, 0)),
        ],
        out_specs=pl.BlockSpec((BM, na), lambda i: (i, 0)),
        out_shape=jax.ShapeDtypeStruct((n, na), F32),
    )(x, wa)


def _combine2(a, b, bias, n):
    mb = pl.cdiv(n, BM)

    def body(a_ref, b_ref, bias_ref, o_ref):
        o_ref[...] = jnp.maximum(a_ref[...] + b_ref[...] + bias_ref[...], 0.0)

    return pl.pallas_call(
        body,
        grid=(mb,),
        in_specs=[
            pl.BlockSpec((BM, D), lambda i: (i, 0)),
            pl.BlockSpec((BM, D), lambda i: (i, 0)),
            pl.BlockSpec((1, D), lambda i: (0, 0)),
        ],
        out_specs=pl.BlockSpec((BM, D), lambda i: (i, 0)),
        out_shape=jax.ShapeDtypeStruct((n, D), F32),
    )(a, b, bias)


def _combine1(a, bias, n):
    mb = pl.cdiv(n, BM)

    def body(a_ref, bias_ref, o_ref):
        o_ref[...] = jnp.maximum(a_ref[...] + bias_ref[...], 0.0)

    return pl.pallas_call(
        body,
        grid=(mb,),
        in_specs=[
            pl.BlockSpec((BM, D), lambda i: (i, 0)),
            pl.BlockSpec((1, D), lambda i: (0, 0)),
        ],
        out_specs=pl.BlockSpec((BM, D), lambda i: (i, 0)),
        out_shape=jax.ShapeDtypeStruct((n, D), F32),
    )(a, bias)


@functools.partial(jax.jit, static_argnums=(4,))
def _sc_pass1(a_src_tab, a_dst_tab, src, dst, nd_pad):
    zfull, ztail = divmod(nd_pad // 16, 128)

    def body(asrc_hbm, adst_hbm, src_hbm, dst_hbm, ex_hbm, s_hbm,
             srcc, dstc, arowA, browA, arowB, browB, exbuf, zbuf, dstbuf,
             semA, semB, s_acc):
        cid = lax.axis_index("c")
        sid = lax.axis_index("s")
        zero16 = jnp.zeros((16,), F32)

        tile_base = (cid * 16 + sid) * TK

        pltpu.sync_copy(src_hbm.at[pl.ds(tile_base, TK)], srcc)
        pltpu.sync_copy(dst_hbm.at[pl.ds(tile_base, TK)], dstc)

        def zb(j, _):
            zbuf[j] = zero16
            return 0
        lax.fori_loop(0, 128, zb, 0)

        rpt = nd_pad // 16
        def zs(j, _):
            pltpu.async_copy(zbuf, s_acc.at[pl.ds(sid * rpt + j * 128, 128)],
                             semA)
            return 0
        lax.fori_loop(0, zfull, zs, 0)
        def zsw(j, _):
            pltpu.make_async_copy(
                zbuf, s_acc.at[pl.ds(sid * rpt + j * 128, 128)], semA).wait()
            return 0
        lax.fori_loop(0, zfull, zsw, 0)
        if ztail:
            pltpu.sync_copy(zbuf.at[pl.ds(0, ztail)],
                            s_acc.at[pl.ds(sid * rpt + zfull * 128, ztail)])
        plsc.subcore_barrier()

        def issue(bi, arow, brow, semx):
            pltpu.async_copy(asrc_hbm.at[srcc.at[pl.ds(bi * 128, 128)]],
                             arow, semx)
            pltpu.async_copy(adst_hbm.at[dstc.at[pl.ds(bi * 128, 128)]],
                             brow, semx)

        def drain(bi, arow, brow, semx):
            pltpu.make_async_copy(asrc_hbm.at[srcc.at[pl.ds(bi * 128, 128)]],
                                  arow, semx).wait()
            pltpu.make_async_copy(adst_hbm.at[dstc.at[pl.ds(bi * 128, 128)]],
                                  brow, semx).wait()

        def compute(bi, arow, brow):
            base = tile_base + bi * 128

            def ew(j, _):
                e = arow[j] + brow[j]
                e = jnp.where(e > 0, e, 0.2 * e)
                exbuf[j] = jnp.exp(e)
                return 0
            lax.fori_loop(0, 128, ew, 0)

            pltpu.sync_copy(exbuf, ex_hbm.at[pl.ds(base, 128)])
            def cpi(v, _):
                dstbuf[pl.ds(v * 16, 16)] = dstc[pl.ds(bi * 128 + v * 16, 16)]
                return 0
            lax.fori_loop(0, 8, cpi, 0)
            pltpu.sync_copy(exbuf, s_acc.at[dstbuf], add=True)

        issue(0, arowA, browA, semA)

        def blk2(t, _):
            bi = t * 2
            drain(bi, arowA, browA, semA)

            @pl.when(bi + 1 < NBLK)
            def _():
                issue(bi + 1, arowB, browB, semB)
            compute(bi, arowA, browA)

            @pl.when(bi + 1 < NBLK)
            def _():
                drain(bi + 1, arowB, browB, semB)

                @pl.when(bi + 2 < NBLK)
                def _():
                    issue(bi + 2, arowA, browA, semA)
                compute(bi + 1, arowB, browB)
            return 0
        lax.fori_loop(0, (NBLK + 1) // 2, blk2, 0)
        plsc.subcore_barrier()

        for t in range(zfull):
            pltpu.async_copy(s_acc.at[pl.ds(sid * rpt + t * 128, 128)],
                             s_hbm.at[cid].at[pl.ds(sid * rpt + t * 128, 128)],
                             semB)
        if ztail:
            pltpu.async_copy(s_acc.at[pl.ds(sid * rpt + zfull * 128, ztail)],
                             s_hbm.at[cid].at[pl.ds(sid * rpt + zfull * 128,
                                                    ztail)],
                             semB)
        for t in range(zfull):
            pltpu.make_async_copy(
                s_acc.at[pl.ds(sid * rpt + t * 128, 128)],
                s_hbm.at[cid].at[pl.ds(sid * rpt + t * 128, 128)],
                semB).wait()
        if ztail:
            pltpu.make_async_copy(
                s_acc.at[pl.ds(sid * rpt + zfull * 128, ztail)],
                s_hbm.at[cid].at[pl.ds(sid * rpt + zfull * 128, ztail)],
                semB).wait()

    kern = pl.kernel(
        body,
        out_type=(
            jax.ShapeDtypeStruct((BP, HA), F32),
            jax.ShapeDtypeStruct((2, nd_pad, HA), F32),
        ),
        mesh=_mesh(),
        scratch_types=[
            pltpu.VMEM((TK,), I32),
            pltpu.VMEM((TK,), I32),
            pltpu.VMEM((128, HA), F32),
            pltpu.VMEM((128, HA), F32),
            pltpu.VMEM((128, HA), F32),
            pltpu.VMEM((128, HA), F32),
            pltpu.VMEM((128, HA), F32),
            pltpu.VMEM((128, HA), F32),
            pltpu.VMEM((128,), I32),
            pltpu.SemaphoreType.DMA,
            pltpu.SemaphoreType.DMA,
            pltpu.VMEM_SHARED((nd_pad, HA), F32),
        ],
        compiler_params=pltpu.CompilerParams(use_tc_tiling_on_sc=False),
    )
    return kern(a_src_tab, a_dst_tab, src, dst)


@functools.partial(jax.jit, static_argnums=(5, 6, 7))
def _sc_pass2(h_stack, src, dst, ex, s_part, hslot, nd_pad, chead):
    nranges = nd_pad // R_ROWS
    rpt2 = R_ROWS // 16
    tk2 = TK * 2

    def body(h_hbm, src_hbm, dst_hbm, ex_hbm, s_hbm, out_hbm,
             dstc, srcc, cb_src, cb_pk,
             gidxA, sidxA, eidxA, scidxA, gidxB, sidxB, eidxB, scidxB,
             rowA, exA, s0A, s1A, rowB, exB, s0B, s1B,
             wbuf, zbuf, semA, semB, acc):
        cid = lax.axis_index("c")
        sid = lax.axis_index("s")
        iota = _iota16()
        tile_base = sid * tk2
        zero16 = jnp.zeros((16,), F32)

        pltpu.sync_copy(dst_hbm.at[pl.ds(tile_base, tk2)], dstc)
        pltpu.sync_copy(src_hbm.at[pl.ds(tile_base, tk2)], srcc)

        def zb(k, _):
            for j in range(16):
                zbuf[k, pl.ds(j * 16, 16)] = zero16
            return 0
        lax.fori_loop(0, 8, zb, 0)

        def mk(cnt, k0, gidx, sidx, eidx, scidx, lo):
            def mkidx(v, _):
                pos = k0 + v * 16
                m = (pos + iota) < cnt
                sv = cb_src[pl.ds(pos, 16)]
                pk = cb_pk[pl.ds(pos, 16)]
                ev = pk & 0x1FFFF
                lv = lax.shift_right_logical(pk, 17)
                gidx[pl.ds(v * 16, 16)] = jnp.where(m, sv, 0)
                sidx[pl.ds(v * 16, 16)] = jnp.where(m, lv + lo, 0)
                eidx[pl.ds(v * 16, 16)] = jnp.where(m, ev, 0)
                scidx[pl.ds(v * 16, 16)] = jnp.where(m, lv, R_ROWS)
                return 0
            lax.fori_loop(0, KB2 // 16, mkidx, 0)

        def issue(gidx, sidx, eidx, rowb, exb, s0b, s1b, semx):
            pltpu.async_copy(h_hbm.at[hslot].at[gidx], rowb, semx)
            pltpu.async_copy(ex_hbm.at[eidx], exb, semx)
            pltpu.async_copy(s_hbm.at[0].at[sidx], s0b, semx)
            pltpu.async_copy(s_hbm.at[1].at[sidx], s1b, semx)

        def drain(gidx, sidx, eidx, rowb, exb, s0b, s1b, semx):
            pltpu.make_async_copy(h_hbm.at[hslot].at[gidx], rowb, semx).wait()
            pltpu.make_async_copy(ex_hbm.at[eidx], exb, semx).wait()
            pltpu.make_async_copy(s_hbm.at[0].at[sidx], s0b, semx).wait()
            pltpu.make_async_copy(s_hbm.at[1].at[sidx], s1b, semx).wait()

        def compute_scatter(rowb, exb, s0b, s1b, scidx):
            def ew(j, _):
                wbuf[pl.ds(j * 16, 16)] = (
                    exb[j] / (s0b[j] + s1b[j] + 1e-16))
                return 0
            lax.fori_loop(0, KB2, ew, 0)

            def rowfn(e2, _):
                wsp = None
                prev_hd = -1
                for j in range(16):
                    hd = (16 * j) // chead
                    if hd != prev_hd:
                        wsp = plsc.load_gather(
                            wbuf, [jnp.full((16,), e2 * 16 + hd, I32)])
                        prev_hd = hd
                    rowb[e2, pl.ds(j * 16, 16)] = (
                        rowb[e2, pl.ds(j * 16, 16)] * wsp)
                return 0
            lax.fori_loop(0, KB2, rowfn, 0)
            pltpu.sync_copy(rowb, acc.at[scidx], add=True)

        def range_body(r, _):
            lo = r * R_ROWS

            @pl.when(lax.rem(r, 2) == cid)
            def _():
                for t in range(rpt2 // 8):
                    pltpu.async_copy(zbuf,
                                     acc.at[pl.ds(sid * rpt2 + t * 8, 8)],
                                     semA)
                for t in range(rpt2 // 8):
                    pltpu.make_async_copy(
                        zbuf, acc.at[pl.ds(sid * rpt2 + t * 8, 8)],
                        semA).wait()
                plsc.subcore_barrier()

                def scan(j, cnt_v):
                    d = dstc[pl.ds(j * 16, 16)]
                    s = srcc[pl.ds(j * 16, 16)]
                    lv = d - lo
                    m = (lv >= 0) & (lv < R_ROWS)
                    pos = jnp.where(m, cnt_v + plsc.cumsum(m.astype(I32)) - 1,
                                    tk2 + 8)
                    plsc.store_scatter(cb_src, [pos], s)
                    eid = (tile_base + j * 16) + iota
                    plsc.store_scatter(cb_pk, [pos],
                                       eid | lax.shift_left(lv, 17))
                    return cnt_v + plsc.all_reduce_population_count(m)
                cnt_v = lax.fori_loop(0, tk2 // 16, scan,
                                      jnp.zeros((16,), I32))
                cnt = cnt_v[0]

                nb = (cnt + (KB2 - 1)) // KB2

                @pl.when(nb > 0)
                def _():
                    mk(cnt, 0, gidxA, sidxA, eidxA, scidxA, lo)
                    issue(gidxA, sidxA, eidxA, rowA, exA, s0A, s1A, semA)

                def proc2(t, _):
                    bb = t * 2

                    drain(gidxA, sidxA, eidxA, rowA, exA, s0A, s1A, semA)

                    @pl.when(bb + 1 < nb)
                    def _():
                        mk(cnt, (bb + 1) * KB2, gidxB, sidxB, eidxB, scidxB,
                           lo)
                        issue(gidxB, sidxB, eidxB, rowB, exB, s0B, s1B, semB)

                    compute_scatter(rowA, exA, s0A, s1A, scidxA)

                    @pl.when(bb + 1 < nb)
                    def _():
                        drain(gidxB, sidxB, eidxB, rowB, exB, s0B, s1B, semB)

                        @pl.when(bb + 2 < nb)
                        def _():
                            mk(cnt, (bb + 2) * KB2, gidxA, sidxA, eidxA,
                               scidxA, lo)
                            issue(gidxA, sidxA, eidxA, rowA, exA, s0A, s1A,
                                  semA)

                        compute_scatter(rowB, exB, s0B, s1B, scidxB)
                    return 0
                lax.fori_loop(0, (nb + 1) // 2, proc2, 0)
                plsc.subcore_barrier()

                pltpu.sync_copy(acc.at[pl.ds(sid * rpt2, rpt2)],
                                out_hbm.at[pl.ds(lo + sid * rpt2, rpt2)])
            return 0
        lax.fori_loop(0, nranges, range_body, 0)

    kern = pl.kernel(
        body,
        out_type=jax.ShapeDtypeStruct((nd_pad, D), F32),
        mesh=_mesh(),
        scratch_types=[
            pltpu.VMEM((tk2,), I32),
            pltpu.VMEM((tk2,), I32),
            pltpu.VMEM((tk2 + 16,), I32),
            pltpu.VMEM((tk2 + 16,), I32),
            pltpu.VMEM((KB2,), I32),
            pltpu.VMEM((KB2,), I32),
            pltpu.VMEM((KB2,), I32),
            pltpu.VMEM((KB2,), I32),
            pltpu.VMEM((KB2,), I32),
            pltpu.VMEM((KB2,), I32),
            pltpu.VMEM((KB2,), I32),
            pltpu.VMEM((KB2,), I32),
            pltpu.VMEM((KB2, D), F32),
            pltpu.VMEM((KB2, HA), F32),
            pltpu.VMEM((KB2, HA), F32),
            pltpu.VMEM((KB2, HA), F32),
            pltpu.VMEM((KB2, D), F32),
            pltpu.VMEM((KB2, HA), F32),
            pltpu.VMEM((KB2, HA), F32),
            pltpu.VMEM((KB2, HA), F32),
            pltpu.VMEM((KB2 * HA,), F32),
            pltpu.VMEM((8, D), F32),
            pltpu.SemaphoreType.DMA,
            pltpu.SemaphoreType.DMA,
            pltpu.VMEM_SHARED((R_ROWS + 8, D), F32),
        ],
        compiler_params=pltpu.CompilerParams(use_tc_tiling_on_sc=False,
                                             needs_layout_passes=False),
    )
    return kern(h_stack, src, dst, ex, s_part)


def _att_fold(p, heads, chead):
    wr = p["W"].reshape(D, heads, chead)
    a_s = jnp.einsum("khc,hc->kh", wr, p["att_src"],
                     precision=jax.lax.Precision.HIGHEST)
    a_d = jnp.einsum("khc,hc->kh", wr, p["att_dst"],
                     precision=jax.lax.Precision.HIGHEST)
    if heads < HA:
        a_s = jnp.pad(a_s, ((0, 0), (0, HA - heads)))
        a_d = jnp.pad(a_d, ((0, 0), (0, HA - heads)))
    return a_s, a_d


def _pad_rows(a, extra=8):
    return jnp.pad(a, ((0, extra), (0, 0)))


def _pad_edges(e, n_dst):
    src = e[0].astype(I32)
    dst = e[1].astype(I32)
    pad = BP - B_EDGE
    src = jnp.concatenate([src, jnp.zeros((pad,), I32)])
    dst = jnp.concatenate([dst, jnp.full((pad,), n_dst, I32)])
    return src, dst


def kernel(x_individual, x_family, params,
           edge_index_individual_child_of_family,
           edge_index_family_parent_of_individual,
           edge_index_individual_spouse_individual):
    n_ind = x_individual.shape[0]
    n_fam = x_family.shape[0]
    ndp_ind = ((n_ind + 8 + R_ROWS - 1) // R_ROWS) * R_ROWS
    ndp_fam = ((n_fam + 8 + R_ROWS - 1) // R_ROWS) * R_ROWS

    s1e, d1e = _pad_edges(edge_index_individual_child_of_family, n_fam)
    s2e, d2e = _pad_edges(edge_index_family_parent_of_individual, n_ind)
    s3e, d3e = _pad_edges(edge_index_individual_spouse_individual, n_ind)

    pe_i = params["emb"]["individual"]
    pe_f = params["emb"]["family"]
    x_i = _mm_stacked(x_individual, pe_i["W"][None], pe_i["b"][None], True)[0]
    x_f = _mm_stacked(x_family, pe_f["W"][None], pe_f["b"][None], True)[0]

    k1 = "individual__child_of__family"
    k2 = "family__parent_of__individual"
    k3 = "individual__spouse__individual"

    for l in range(4):
        concat = l < 3
        heads = 8 if concat else 1
        chead = D // heads
        lp = params["convs"][l]
        p1, p2, p3 = lp[k1], lp[k2], lp[k3]

        u_ind = _mm_stacked(x_i, jnp.stack([p1["W"], p3["W"]]),
                            jnp.zeros((2, D), F32), False)
        u_fam = _mm_stacked(x_f, p2["W"][None], jnp.zeros((1, D), F32), False)

        a1s, a1d = _att_fold(p1, heads, chead)
        a2s, a2d = _att_fold(p2, heads, chead)
        a3s, a3d = _att_fold(p3, heads, chead)
        wa_ind = jnp.concatenate([a1s, a2d, a3s, a3d], axis=1)
        wa_fam = jnp.concatenate([a1d, a2s], axis=1)
        ai = _mm_thin(x_i, wa_ind)
        af = _mm_thin(x_f, wa_fam)

        t1s = _pad_rows(ai[:, 0:16])
        t2d = _pad_rows(ai[:, 16:32])
        t3s = _pad_rows(ai[:, 32:48])
        t3d = _pad_rows(ai[:, 48:64])
        t1d = _pad_rows(af[:, 0:16])
        t2s = _pad_rows(af[:, 16:32])

        ex1, sp1 = _sc_pass1(t1s, t1d, s1e, d1e, ndp_fam)
        ex2, sp2 = _sc_pass1(t2s, t2d, s2e, d2e, ndp_ind)
        ex3, sp3 = _sc_pass1(t3s, t3d, s3e, d3e, ndp_ind)

        o1 = _sc_pass2(u_ind, s1e, d1e, ex1, sp1, 0, ndp_fam, chead)
        o2 = _sc_pass2(u_fam, s2e, d2e, ex2, sp2, 0, ndp_ind, chead)
        o3 = _sc_pass2(u_ind, s3e, d3e, ex3, sp3, 1, ndp_ind, chead)

        x_f = _combine1(o1, p1["bias"][None], n_fam)
        x_i = _combine2(o2, o3, (p2["bias"] + p3["bias"])[None], n_ind)

    pf = params["pred"]["father"]
    pm = params["pred"]["mother"]
    pred = _mm_stacked(x_i, jnp.stack([pf["W"], pm["W"]]),
                       jnp.stack([pf["b"], pm["b"]]), False)
    return (x_i, x_f, pred[0], pred[1])


# speedup vs baseline: 14.5827x; 1.0006x over previous
import functools

import jax
import jax.numpy as jnp
from jax import lax
from jax.experimental import pallas as pl
from jax.experimental.pallas import tpu as pltpu
from jax.experimental.pallas import tpu_sc as plsc

F32 = jnp.float32
I32 = jnp.int32

D = 256
HA = 16
B_EDGE = 100000
NTILE = 32
TK = 3200
BP = NTILE * TK
NBLK = TK // 128
R_ROWS = 3584
KB2 = 64
BM = 512


def _mesh():
    return plsc.VectorSubcoreMesh(core_axis_name="c", subcore_axis_name="s")


def _iota16():
    return jax.lax.iota(I32, 16)


def _mm_stacked(x, w_stack, bias, relu):
    n = x.shape[0]
    s_chunks = w_stack.shape[0]
    mb = pl.cdiv(n, BM)

    def body(x_ref, w_ref, b_ref, o_ref):
        acc = jnp.dot(x_ref[...], w_ref[0], preferred_element_type=F32)
        acc = acc + b_ref[0]
        if relu:
            acc = jnp.maximum(acc, 0.0)
        o_ref[0] = acc

    return pl.pallas_call(
        body,
        grid=(mb, s_chunks),
        in_specs=[
            pl.BlockSpec((BM, D), lambda i, j: (i, 0)),
            pl.BlockSpec((1, D, D), lambda i, j: (j, 0, 0)),
            pl.BlockSpec((1, 1, D), lambda i, j: (j, 0, 0)),
        ],
        out_specs=pl.BlockSpec((1, BM, D), lambda i, j: (j, i, 0)),
        out_shape=jax.ShapeDtypeStruct((s_chunks, n, D), F32),
    )(x, w_stack, bias[:, None, :])


def _mm_thin(x, wa):
    n = x.shape[0]
    na = wa.shape[1]
    mb = pl.cdiv(n, BM)

    def body(x_ref, w_ref, o_ref):
        o_ref[...] = jnp.dot(x_ref[...], w_ref[...], preferred_element_type=F32)

    return pl.pallas_call(
        body,
        grid=(mb,),
        in_specs=[
            pl.BlockSpec((BM, D), lambda i: (i, 0)),
            pl.BlockSpec((D, na), lambda i: (0, 0)),
        ],
        out_specs=pl.BlockSpec((BM, na), lambda i: (i, 0)),
        out_shape=jax.ShapeDtypeStruct((n, na), F32),
    )(x, wa)


def _combine2(a, b, bias, n):
    mb = pl.cdiv(n, BM)

    def body(a_ref, b_ref, bias_ref, o_ref):
        o_ref[...] = jnp.maximum(a_ref[...] + b_ref[...] + bias_ref[...], 0.0)

    return pl.pallas_call(
        body,
        grid=(mb,),
        in_specs=[
            pl.BlockSpec((BM, D), lambda i: (i, 0)),
            pl.BlockSpec((BM, D), lambda i: (i, 0)),
            pl.BlockSpec((1, D), lambda i: (0, 0)),
        ],
        out_specs=pl.BlockSpec((BM, D), lambda i: (i, 0)),
        out_shape=jax.ShapeDtypeStruct((n, D), F32),
    )(a, b, bias)


def _combine1(a, bias, n):
    mb = pl.cdiv(n, BM)

    def body(a_ref, bias_ref, o_ref):
        o_ref[...] = jnp.maximum(a_ref[...] + bias_ref[...], 0.0)

    return pl.pallas_call(
        body,
        grid=(mb,),
        in_specs=[
            pl.BlockSpec((BM, D), lambda i: (i, 0)),
            pl.BlockSpec((1, D), lambda i: (0, 0)),
        ],
        out_specs=pl.BlockSpec((BM, D), lambda i: (i, 0)),
        out_shape=jax.ShapeDtypeStruct((n, D), F32),
    )(a, bias)


@functools.partial(jax.jit, static_argnums=(4,))
def _sc_pass1(a_src_tab, a_dst_tab, src, dst, nd_pad):
    zfull, ztail = divmod(nd_pad // 16, 128)

    def body(asrc_hbm, adst_hbm, src_hbm, dst_hbm, ex_hbm, s_hbm,
             srcc, dstc, arowA, browA, arowB, browB, exbuf, zbuf, dstbuf,
             semA, semB, s_acc):
        cid = lax.axis_index("c")
        sid = lax.axis_index("s")
        zero16 = jnp.zeros((16,), F32)

        tile_base = (cid * 16 + sid) * TK

        pltpu.sync_copy(src_hbm.at[pl.ds(tile_base, TK)], srcc)
        pltpu.sync_copy(dst_hbm.at[pl.ds(tile_base, TK)], dstc)

        def zb(j, _):
            zbuf[j] = zero16
            return 0
        lax.fori_loop(0, 128, zb, 0)

        rpt = nd_pad // 16
        def zs(j, _):
            pltpu.async_copy(zbuf, s_acc.at[pl.ds(sid * rpt + j * 128, 128)],
                             semA)
            return 0
        lax.fori_loop(0, zfull, zs, 0)
        def zsw(j, _):
            pltpu.make_async_copy(
                zbuf, s_acc.at[pl.ds(sid * rpt + j * 128, 128)], semA).wait()
            return 0
        lax.fori_loop(0, zfull, zsw, 0)
        if ztail:
            pltpu.sync_copy(zbuf.at[pl.ds(0, ztail)],
                            s_acc.at[pl.ds(sid * rpt + zfull * 128, ztail)])
        plsc.subcore_barrier()

        def issue(bi, arow, brow, semx):
            pltpu.async_copy(asrc_hbm.at[srcc.at[pl.ds(bi * 128, 128)]],
                             arow, semx)
            pltpu.async_copy(adst_hbm.at[dstc.at[pl.ds(bi * 128, 128)]],
                             brow, semx)

        def drain(bi, arow, brow, semx):
            pltpu.make_async_copy(asrc_hbm.at[srcc.at[pl.ds(bi * 128, 128)]],
                                  arow, semx).wait()
            pltpu.make_async_copy(adst_hbm.at[dstc.at[pl.ds(bi * 128, 128)]],
                                  brow, semx).wait()

        def compute(bi, arow, brow):
            base = tile_base + bi * 128

            def ew(j, _):
                e = arow[j] + brow[j]
                e = jnp.where(e > 0, e, 0.2 * e)
                exbuf[j] = jnp.exp(e)
                return 0
            lax.fori_loop(0, 128, ew, 0)

            pltpu.sync_copy(exbuf, ex_hbm.at[pl.ds(base, 128)])
            def cpi(v, _):
                dstbuf[pl.ds(v * 16, 16)] = dstc[pl.ds(bi * 128 + v * 16, 16)]
                return 0
            lax.fori_loop(0, 8, cpi, 0)
            pltpu.sync_copy(exbuf, s_acc.at[dstbuf], add=True)

        issue(0, arowA, browA, semA)

        def blk2(t, _):
            bi = t * 2
            drain(bi, arowA, browA, semA)

            @pl.when(bi + 1 < NBLK)
            def _():
                issue(bi + 1, arowB, browB, semB)
            compute(bi, arowA, browA)

            @pl.when(bi + 1 < NBLK)
            def _():
                drain(bi + 1, arowB, browB, semB)

                @pl.when(bi + 2 < NBLK)
                def _():
                    issue(bi + 2, arowA, browA, semA)
                compute(bi + 1, arowB, browB)
            return 0
        lax.fori_loop(0, (NBLK + 1) // 2, blk2, 0)
        plsc.subcore_barrier()

        for t in range(zfull):
            pltpu.async_copy(s_acc.at[pl.ds(sid * rpt + t * 128, 128)],
                             s_hbm.at[cid].at[pl.ds(sid * rpt + t * 128, 128)],
                             semB)
        if ztail:
            pltpu.async_copy(s_acc.at[pl.ds(sid * rpt + zfull * 128, ztail)],
                             s_hbm.at[cid].at[pl.ds(sid * rpt + zfull * 128,
                                                    ztail)],
                             semB)
        for t in range(zfull):
            pltpu.make_async_copy(
                s_acc.at[pl.ds(sid * rpt + t * 128, 128)],
                s_hbm.at[cid].at[pl.ds(sid * rpt + t * 128, 128)],
                semB).wait()
        if ztail:
            pltpu.make_async_copy(
                s_acc.at[pl.ds(sid * rpt + zfull * 128, ztail)],
                s_hbm.at[cid].at[pl.ds(sid * rpt + zfull * 128, ztail)],
                semB).wait()

    kern = pl.kernel(
        body,
        out_type=(
            jax.ShapeDtypeStruct((BP, HA), F32),
            jax.ShapeDtypeStruct((2, nd_pad, HA), F32),
        ),
        mesh=_mesh(),
        scratch_types=[
            pltpu.VMEM((TK,), I32),
            pltpu.VMEM((TK,), I32),
            pltpu.VMEM((128, HA), F32),
            pltpu.VMEM((128, HA), F32),
            pltpu.VMEM((128, HA), F32),
            pltpu.VMEM((128, HA), F32),
            pltpu.VMEM((128, HA), F32),
            pltpu.VMEM((128, HA), F32),
            pltpu.VMEM((128,), I32),
            pltpu.SemaphoreType.DMA,
            pltpu.SemaphoreType.DMA,
            pltpu.VMEM_SHARED((nd_pad, HA), F32),
        ],
        compiler_params=pltpu.CompilerParams(use_tc_tiling_on_sc=False),
    )
    return kern(a_src_tab, a_dst_tab, src, dst)


@functools.partial(jax.jit, static_argnums=(5, 6, 7))
def _sc_pass2(h_stack, src, dst, ex, s_part, hslot, nd_pad, chead):
    nranges = nd_pad // R_ROWS
    rpt2 = R_ROWS // 16
    tk2 = TK * 2

    def body(h_hbm, src_hbm, dst_hbm, ex_hbm, s_hbm, out_hbm,
             dstc, srcc, cb_src, cb_pk,
             gidxA, sidxA, eidxA, scidxA, gidxB, sidxB, eidxB, scidxB,
             rowA, exA, s0A, s1A, rowB, exB, s0B, s1B,
             wbuf, zbuf, semA, semB, semSA, semSB, acc):
        cid = lax.axis_index("c")
        sid = lax.axis_index("s")
        iota = _iota16()
        tile_base = sid * tk2
        zero16 = jnp.zeros((16,), F32)

        pltpu.sync_copy(dst_hbm.at[pl.ds(tile_base, tk2)], dstc)
        pltpu.sync_copy(src_hbm.at[pl.ds(tile_base, tk2)], srcc)

        def zb(k, _):
            for j in range(16):
                zbuf[k, pl.ds(j * 16, 16)] = zero16
            return 0
        lax.fori_loop(0, 8, zb, 0)

        def mk(cnt, k0, gidx, sidx, eidx, scidx, lo):
            def mkidx(v, _):
                pos = k0 + v * 16
                m = (pos + iota) < cnt
                sv = cb_src[pl.ds(pos, 16)]
                pk = cb_pk[pl.ds(pos, 16)]
                ev = pk & 0x1FFFF
                lv = lax.shift_right_logical(pk, 17)
                gidx[pl.ds(v * 16, 16)] = jnp.where(m, sv, 0)
                sidx[pl.ds(v * 16, 16)] = jnp.where(m, lv + lo, 0)
                eidx[pl.ds(v * 16, 16)] = jnp.where(m, ev, 0)
                scidx[pl.ds(v * 16, 16)] = jnp.where(m, lv, R_ROWS)
                return 0
            lax.fori_loop(0, KB2 // 16, mkidx, 0)

        def issue(gidx, sidx, eidx, rowb, exb, s0b, s1b, semx):
            pltpu.async_copy(h_hbm.at[hslot].at[gidx], rowb, semx)
            pltpu.async_copy(ex_hbm.at[eidx], exb, semx)
            pltpu.async_copy(s_hbm.at[0].at[sidx], s0b, semx)
            pltpu.async_copy(s_hbm.at[1].at[sidx], s1b, semx)

        def drain(gidx, sidx, eidx, rowb, exb, s0b, s1b, semx):
            pltpu.make_async_copy(h_hbm.at[hslot].at[gidx], rowb, semx).wait()
            pltpu.make_async_copy(ex_hbm.at[eidx], exb, semx).wait()
            pltpu.make_async_copy(s_hbm.at[0].at[sidx], s0b, semx).wait()
            pltpu.make_async_copy(s_hbm.at[1].at[sidx], s1b, semx).wait()

        def compute_scatter(rowb, exb, s0b, s1b, scidx, semsx):
            def ew(j, _):
                wbuf[pl.ds(j * 16, 16)] = (
                    exb[j] / (s0b[j] + s1b[j] + 1e-16))
                return 0
            lax.fori_loop(0, KB2, ew, 0)

            def rowfn(e2, _):
                wsp = None
                prev_hd = -1
                for j in range(16):
                    hd = (16 * j) // chead
                    if hd != prev_hd:
                        wsp = plsc.load_gather(
                            wbuf, [jnp.full((16,), e2 * 16 + hd, I32)])
                        prev_hd = hd
                    rowb[e2, pl.ds(j * 16, 16)] = (
                        rowb[e2, pl.ds(j * 16, 16)] * wsp)
                return 0
            lax.fori_loop(0, KB2, rowfn, 0)
            pltpu.async_copy(rowb, acc.at[scidx], semsx, add=True)

        def drain_scatter(rowb, scidx, semsx):
            pltpu.make_async_copy(rowb, acc.at[scidx], semsx).wait()

        def range_body(r, _):
            lo = r * R_ROWS

            @pl.when(lax.rem(r, 2) == cid)
            def _():
                for t in range(rpt2 // 8):
                    pltpu.async_copy(zbuf,
                                     acc.at[pl.ds(sid * rpt2 + t * 8, 8)],
                                     semA)
                for t in range(rpt2 // 8):
                    pltpu.make_async_copy(
                        zbuf, acc.at[pl.ds(sid * rpt2 + t * 8, 8)],
                        semA).wait()
                plsc.subcore_barrier()

                def scan(j, cnt_v):
                    d = dstc[pl.ds(j * 16, 16)]
                    s = srcc[pl.ds(j * 16, 16)]
                    lv = d - lo
                    m = (lv >= 0) & (lv < R_ROWS)
                    pos = jnp.where(m, cnt_v + plsc.cumsum(m.astype(I32)) - 1,
                                    tk2 + 8)
                    plsc.store_scatter(cb_src, [pos], s)
                    eid = (tile_base + j * 16) + iota
                    plsc.store_scatter(cb_pk, [pos],
                                       eid | lax.shift_left(lv, 17))
                    return cnt_v + plsc.all_reduce_population_count(m)
                cnt_v = lax.fori_loop(0, tk2 // 16, scan,
                                      jnp.zeros((16,), I32))
                cnt = cnt_v[0]

                nb = (cnt + (KB2 - 1)) // KB2

                @pl.when(nb > 0)
                def _():
                    mk(cnt, 0, gidxA, sidxA, eidxA, scidxA, lo)
                    issue(gidxA, sidxA, eidxA, rowA, exA, s0A, s1A, semA)

                def proc2(t, _):
                    bb = t * 2

                    drain(gidxA, sidxA, eidxA, rowA, exA, s0A, s1A, semA)

                    @pl.when(bb + 1 < nb)
                    def _():
                        @pl.when(bb >= 2)
                        def _():
                            drain_scatter(rowB, scidxB, semSB)
                        mk(cnt, (bb + 1) * KB2, gidxB, sidxB, eidxB, scidxB,
                           lo)
                        issue(gidxB, sidxB, eidxB, rowB, exB, s0B, s1B, semB)

                    compute_scatter(rowA, exA, s0A, s1A, scidxA, semSA)

                    @pl.when(bb + 1 < nb)
                    def _():
                        drain(gidxB, sidxB, eidxB, rowB, exB, s0B, s1B, semB)

                        @pl.when(bb + 2 < nb)
                        def _():
                            drain_scatter(rowA, scidxA, semSA)
                            mk(cnt, (bb + 2) * KB2, gidxA, sidxA, eidxA,
                               scidxA, lo)
                            issue(gidxA, sidxA, eidxA, rowA, exA, s0A, s1A,
                                  semA)

                        compute_scatter(rowB, exB, s0B, s1B, scidxB, semSB)
                    return 0
                lax.fori_loop(0, (nb + 1) // 2, proc2, 0)

                @pl.when(nb >= 2)
                def _():
                    drain_scatter(rowA, scidxA, semSA)
                    drain_scatter(rowB, scidxB, semSB)

                @pl.when(nb == 1)
                def _():
                    drain_scatter(rowA, scidxA, semSA)
                plsc.subcore_barrier()

                pltpu.sync_copy(acc.at[pl.ds(sid * rpt2, rpt2)],
                                out_hbm.at[pl.ds(lo + sid * rpt2, rpt2)])
            return 0
        lax.fori_loop(0, nranges, range_body, 0)

    kern = pl.kernel(
        body,
        out_type=jax.ShapeDtypeStruct((nd_pad, D), F32),
        mesh=_mesh(),
        scratch_types=[
            pltpu.VMEM((tk2,), I32),
            pltpu.VMEM((tk2,), I32),
            pltpu.VMEM((tk2 + 16,), I32),
            pltpu.VMEM((tk2 + 16,), I32),
            pltpu.VMEM((KB2,), I32),
            pltpu.VMEM((KB2,), I32),
            pltpu.VMEM((KB2,), I32),
            pltpu.VMEM((KB2,), I32),
            pltpu.VMEM((KB2,), I32),
            pltpu.VMEM((KB2,), I32),
            pltpu.VMEM((KB2,), I32),
            pltpu.VMEM((KB2,), I32),
            pltpu.VMEM((KB2, D), F32),
            pltpu.VMEM((KB2, HA), F32),
            pltpu.VMEM((KB2, HA), F32),
            pltpu.VMEM((KB2, HA), F32),
            pltpu.VMEM((KB2, D), F32),
            pltpu.VMEM((KB2, HA), F32),
            pltpu.VMEM((KB2, HA), F32),
            pltpu.VMEM((KB2, HA), F32),
            pltpu.VMEM((KB2 * HA,), F32),
            pltpu.VMEM((8, D), F32),
            pltpu.SemaphoreType.DMA,
            pltpu.SemaphoreType.DMA,
            pltpu.SemaphoreType.DMA,
            pltpu.SemaphoreType.DMA,
            pltpu.VMEM_SHARED((R_ROWS + 8, D), F32),
        ],
        compiler_params=pltpu.CompilerParams(use_tc_tiling_on_sc=False,
                                             needs_layout_passes=False),
    )
    return kern(h_stack, src, dst, ex, s_part)


def _att_fold(p, heads, chead):
    wr = p["W"].reshape(D, heads, chead)
    a_s = jnp.einsum("khc,hc->kh", wr, p["att_src"],
                     precision=jax.lax.Precision.HIGHEST)
    a_d = jnp.einsum("khc,hc->kh", wr, p["att_dst"],
                     precision=jax.lax.Precision.HIGHEST)
    if heads < HA:
        a_s = jnp.pad(a_s, ((0, 0), (0, HA - heads)))
        a_d = jnp.pad(a_d, ((0, 0), (0, HA - heads)))
    return a_s, a_d


def _pad_rows(a, extra=8):
    return jnp.pad(a, ((0, extra), (0, 0)))


def _pad_edges(e, n_dst):
    src = e[0].astype(I32)
    dst = e[1].astype(I32)
    pad = BP - B_EDGE
    src = jnp.concatenate([src, jnp.zeros((pad,), I32)])
    dst = jnp.concatenate([dst, jnp.full((pad,), n_dst, I32)])
    return src, dst


def kernel(x_individual, x_family, params,
           edge_index_individual_child_of_family,
           edge_index_family_parent_of_individual,
           edge_index_individual_spouse_individual):
    n_ind = x_individual.shape[0]
    n_fam = x_family.shape[0]
    ndp_ind = ((n_ind + 8 + R_ROWS - 1) // R_ROWS) * R_ROWS
    ndp_fam = ((n_fam + 8 + R_ROWS - 1) // R_ROWS) * R_ROWS

    s1e, d1e = _pad_edges(edge_index_individual_child_of_family, n_fam)
    s2e, d2e = _pad_edges(edge_index_family_parent_of_individual, n_ind)
    s3e, d3e = _pad_edges(edge_index_individual_spouse_individual, n_ind)

    pe_i = params["emb"]["individual"]
    pe_f = params["emb"]["family"]
    x_i = _mm_stacked(x_individual, pe_i["W"][None], pe_i["b"][None], True)[0]
    x_f = _mm_stacked(x_family, pe_f["W"][None], pe_f["b"][None], True)[0]

    k1 = "individual__child_of__family"
    k2 = "family__parent_of__individual"
    k3 = "individual__spouse__individual"

    for l in range(4):
        concat = l < 3
        heads = 8 if concat else 1
        chead = D // heads
        lp = params["convs"][l]
        p1, p2, p3 = lp[k1], lp[k2], lp[k3]

        u_ind = _mm_stacked(x_i, jnp.stack([p1["W"], p3["W"]]),
                            jnp.zeros((2, D), F32), False)
        u_fam = _mm_stacked(x_f, p2["W"][None], jnp.zeros((1, D), F32), False)

        a1s, a1d = _att_fold(p1, heads, chead)
        a2s, a2d = _att_fold(p2, heads, chead)
        a3s, a3d = _att_fold(p3, heads, chead)
        wa_ind = jnp.concatenate([a1s, a2d, a3s, a3d], axis=1)
        wa_fam = jnp.concatenate([a1d, a2s], axis=1)
        ai = _mm_thin(x_i, wa_ind)
        af = _mm_thin(x_f, wa_fam)

        t1s = _pad_rows(ai[:, 0:16])
        t2d = _pad_rows(ai[:, 16:32])
        t3s = _pad_rows(ai[:, 32:48])
        t3d = _pad_rows(ai[:, 48:64])
        t1d = _pad_rows(af[:, 0:16])
        t2s = _pad_rows(af[:, 16:32])

        ex1, sp1 = _sc_pass1(t1s, t1d, s1e, d1e, ndp_fam)
        ex2, sp2 = _sc_pass1(t2s, t2d, s2e, d2e, ndp_ind)
        ex3, sp3 = _sc_pass1(t3s, t3d, s3e, d3e, ndp_ind)

        o1 = _sc_pass2(u_ind, s1e, d1e, ex1, sp1, 0, ndp_fam, chead)
        o2 = _sc_pass2(u_fam, s2e, d2e, ex2, sp2, 0, ndp_ind, chead)
        o3 = _sc_pass2(u_ind, s3e, d3e, ex3, sp3, 1, ndp_ind, chead)

        x_f = _combine1(o1, p1["bias"][None], n_fam)
        x_i = _combine2(o2, o3, (p2["bias"] + p3["bias"])[None], n_ind)

    pf = params["pred"]["father"]
    pm = params["pred"]["mother"]
    pred = _mm_stacked(x_i, jnp.stack([pf["W"], pm["W"]]),
                       jnp.stack([pf["b"], pm["b"]]), False)
    return (x_i, x_f, pred[0], pred[1])
